# baseline (speedup 1.0000x reference)
import functools
import jax
import jax.numpy as jnp
from jax import lax
from jax.experimental import pallas as pl
from jax.experimental.pallas import tpu as pltpu
from jax.experimental.pallas import tpu_sc as plsc

N = 10000
E = 320000
MP = 10240
RW = 64
NR = 157
NRJ = 5
NWK = 32
NS = 16
SR = NRJ * RW
NPAD = MP
FD = 768
NV = FD // 16
CAP = E + 2048
CE = 800
CL = 1024
G = 32
NGG = CL // G
FB = 256
LB = FB + 32
BM = 512
F32 = jnp.float32
I32 = jnp.int32


def _pad2(x, m, n):
    M, Nc = x.shape
    return jnp.pad(x, ((0, m - M), (0, n - Nc)))


def _mm_body(x_ref, w_ref, b_ref, o_ref, *, act):
    acc = jnp.dot(x_ref[...], w_ref[...], preferred_element_type=F32)
    acc = acc + b_ref[...]
    if act == "lrelu":
        acc = jnp.where(acc > 0, acc, 0.2 * acc)
    o_ref[...] = acc


def _mm(x, w, b=None, act=None):
    M, K = x.shape
    K2, Nc = w.shape
    assert K == K2 and M % BM == 0, (x.shape, w.shape)
    if b is None:
        b = jnp.zeros((Nc,), F32)
    b2 = jnp.pad(b, (0, Nc - b.shape[0])).reshape(1, Nc)
    return pl.pallas_call(
        functools.partial(_mm_body, act=act),
        grid=(M // BM,),
        in_specs=[
            pl.BlockSpec((BM, K), lambda i: (i, 0)),
            pl.BlockSpec((K, Nc), lambda i: (0, 0)),
            pl.BlockSpec((1, Nc), lambda i: (0, 0)),
        ],
        out_specs=pl.BlockSpec((BM, Nc), lambda i: (i, 0)),
        out_shape=jax.ShapeDtypeStruct((M, Nc), F32),
    )(x, w, b2)


def _mm_gat_body(x_ref, w_ref, wa_ref, o_ref, os_ref):
    acc = jnp.dot(x_ref[...], w_ref[...], preferred_element_type=F32)
    o_ref[...] = acc
    os_ref[...] = jnp.dot(acc, wa_ref[...], preferred_element_type=F32)


def _mm_gat(x, w, wa):
    return pl.pallas_call(
        _mm_gat_body,
        grid=(MP // BM,),
        in_specs=[
            pl.BlockSpec((BM, FD), lambda i: (i, 0)),
            pl.BlockSpec((FD, FD), lambda i: (0, 0)),
            pl.BlockSpec((FD, 128), lambda i: (0, 0)),
        ],
        out_specs=[
            pl.BlockSpec((BM, FD), lambda i: (i, 0)),
            pl.BlockSpec((BM, 128), lambda i: (i, 0)),
        ],
        out_shape=[
            jax.ShapeDtypeStruct((MP, FD), F32),
            jax.ShapeDtypeStruct((MP, 128), F32),
        ],
    )(x, w, wa)


def _mulv_body(h_ref, w_ref, b_ref, mu_ref, lv_ref):
    acc = jnp.dot(h_ref[...], w_ref[...], preferred_element_type=F32) + b_ref[...]
    mu_ref[...] = acc[:, 0:128]
    lv_ref[...] = acc[:, 128:256]


def _mm_mulv(h, w, b):
    return pl.pallas_call(
        _mulv_body,
        grid=(MP // BM,),
        in_specs=[
            pl.BlockSpec((BM, 384), lambda i: (i, 0)),
            pl.BlockSpec((384, 256), lambda i: (0, 0)),
            pl.BlockSpec((1, 256), lambda i: (0, 0)),
        ],
        out_specs=[
            pl.BlockSpec((BM, 128), lambda i: (i, 0)),
            pl.BlockSpec((BM, 128), lambda i: (i, 0)),
        ],
        out_shape=[
            jax.ShapeDtypeStruct((MP, 128), F32),
            jax.ShapeDtypeStruct((MP, 128), F32),
        ],
    )(h, w, b.reshape(1, 256))


def _z_body(mu_ref, lv_ref, e_ref, z_ref):
    lv = 0.5 * lv_ref[...]
    std = jnp.where(lv > 0, lv, jnp.exp(lv) - 1.0) + (1.0 + 1e-5)
    z_ref[...] = mu_ref[...] + std * e_ref[...]


def _z_kernel(mu, lv, eps):
    return pl.pallas_call(
        _z_body,
        grid=(MP // BM,),
        in_specs=[pl.BlockSpec((BM, 128), lambda i: (i, 0))] * 3,
        out_specs=pl.BlockSpec((BM, 128), lambda i: (i, 0)),
        out_shape=jax.ShapeDtypeStruct((MP, 128), F32),
    )(mu, lv, eps)


def _make_sc(bucketize):
    mesh = plsc.VectorSubcoreMesh(core_axis_name="c", subcore_axis_name="s")
    out_type = [jax.ShapeDtypeStruct((NPAD, FD), F32)]
    if bucketize:
        out_type += [
            jax.ShapeDtypeStruct((NWK * NRJ * CAP,), I32),
            jax.ShapeDtypeStruct((NWK * NRJ * CAP,), I32),
            jax.ShapeDtypeStruct((NWK * NRJ * CAP,), F32),
            jax.ShapeDtypeStruct((NWK * NRJ * 16,), I32),
        ]
    scratch = [
        pltpu.VMEM((NPAD,), F32),
        pltpu.VMEM((SR,), F32),
        pltpu.VMEM((SR,), F32),
        pltpu.VMEM((16, SR), F32),
        pltpu.VMEM((CL,), I32),
        pltpu.VMEM((CL,), I32),
        pltpu.VMEM((CL,), F32),
        pltpu.VMEM((CL,), I32),
        pltpu.VMEM((CL,), F32),
        pltpu.VMEM((CL,), I32),
        pltpu.VMEM((RW, FD), F32),
        pltpu.VMEM((G, FD), F32),
        pltpu.VMEM((G, FD), F32),
        pltpu.VMEM((RW,), F32),
        pltpu.VMEM((16,), F32),
        pltpu.VMEM((16,), I32),
        pltpu.SemaphoreType.DMA,
        pltpu.SemaphoreType.DMA,
    ]
    if bucketize:
        for _ in range(NRJ):
            scratch += [pltpu.VMEM((LB,), I32), pltpu.VMEM((LB,), I32),
                        pltpu.VMEM((LB,), F32)]

    def body(*refs):
        if bucketize:
            (src_h, dst_h, ew_h, ssrc_h, sdst_h, sn_h, ae_h, hw_h,
             out_h, bs_h, bd_h, be_h, cnt_h,
             ssrc_t, sdst_l, sloc_t, sums_t, c1, c2, c3, ssan, asan, dsan,
             outblk, stgA, stgB, snloc, aev, cbuf, semA, semB,
             *lbufs) = refs
            lsrc = [lbufs[3 * j] for j in range(NRJ)]
            ldst = [lbufs[3 * j + 1] for j in range(NRJ)]
            lew = [lbufs[3 * j + 2] for j in range(NRJ)]
        else:
            (bs_h, bd_h, be_h, cnt_h, ssrc_h, sdst_h, sn_h, ae_h, hw_h,
             out_h,
             ssrc_t, sdst_l, sloc_t, sums_t, c1, c2, c3, ssan, asan, dsan,
             outblk, stgA, stgB, snloc, aev, cbuf, semA, semB) = refs

        wid = lax.axis_index("c") * NS + lax.axis_index("s")
        base = wid * SR
        lane = lax.iota(I32, 16)
        zv = jnp.zeros((16,), F32)

        pltpu.sync_copy(ssrc_h, ssrc_t)
        pltpu.sync_copy(ae_h, aev)
        ae = aev[...][0]
        pltpu.sync_copy(sdst_h.at[pl.ds(pl.multiple_of(base, SR), SR)], sdst_l)

        def _zs(i, _):
            for c in range(SR // 16):
                sums_t[i, pl.ds(c * 16, 16)] = zv
            return 0
        lax.fori_loop(0, 16, _zs, 0)

        if bucketize:
            def chunk_body(ci, carry):
                pltpu.sync_copy(src_h.at[pl.ds(pl.multiple_of(ci * CE, 8), CE)],
                                c1.at[pl.ds(0, CE)])
                pltpu.sync_copy(dst_h.at[pl.ds(pl.multiple_of(ci * CE, 8), CE)],
                                c2.at[pl.ds(0, CE)])
                pltpu.sync_copy(ew_h.at[pl.ds(pl.multiple_of(ci * CE, 8), CE)],
                                c3.at[pl.ds(0, CE)])

                def g_body(g, cy):
                    s16 = c1[pl.ds(g * 16, 16)]
                    d16 = c2[pl.ds(g * 16, 16)]
                    e16 = c3[pl.ds(g * 16, 16)]
                    cidx = d16 - base
                    match = (cidx >= 0) & (cidx < SR)
                    cidx_s = jnp.where(match, cidx, 0)
                    which = jnp.right_shift(cidx_s, 6)
                    sv = plsc.load_gather(ssrc_t, [s16])
                    dv = plsc.load_gather(sdst_l, [cidx_s])
                    ee = sv + dv + e16 * ae
                    ee = jnp.where(ee > 0, ee, ee * 0.2)
                    ex = jnp.exp(ee)
                    plsc.addupdate_scatter(sums_t, [lane, cidx_s], ex,
                                           mask=match)
                    out = []
                    for j in range(NRJ):
                        fj, wj = cy[j], cy[NRJ + j]
                        mj = match & (which == j)
                        plsc.store_compressed(lsrc[j].at[pl.ds(fj, 16)], s16,
                                              mask=mj)
                        plsc.store_compressed(ldst[j].at[pl.ds(fj, 16)], d16,
                                              mask=mj)
                        plsc.store_compressed(lew[j].at[pl.ds(fj, 16)], e16,
                                              mask=mj)
                        fj = fj + plsc.all_reduce_population_count(mj)[0]
                        do = fj >= FB

                        @pl.when(do)
                        def _(j=j, wj=wj):
                            row = wid * NRJ + j
                            pltpu.sync_copy(
                                lsrc[j].at[pl.ds(0, FB)],
                                bs_h.at[pl.ds(pl.multiple_of(row * CAP + wj, FB), FB)])
                            pltpu.sync_copy(
                                ldst[j].at[pl.ds(0, FB)],
                                bd_h.at[pl.ds(pl.multiple_of(row * CAP + wj, FB), FB)])
                            pltpu.sync_copy(
                                lew[j].at[pl.ds(0, FB)],
                                be_h.at[pl.ds(pl.multiple_of(row * CAP + wj, FB), FB)])
                            t1 = lsrc[j][pl.ds(FB, 16)]
                            lsrc[j][pl.ds(0, 16)] = t1
                            t2 = ldst[j][pl.ds(FB, 16)]
                            ldst[j][pl.ds(0, 16)] = t2
                            t3 = lew[j][pl.ds(FB, 16)]
                            lew[j][pl.ds(0, 16)] = t3
                        out.append((jnp.where(do, fj - FB, fj),
                                    jnp.where(do, wj + FB, wj)))
                    return tuple([o[0] for o in out] + [o[1] for o in out])
                return lax.fori_loop(0, CE // 16, g_body, carry)

            z0 = jnp.zeros((), I32)
            fw = lax.fori_loop(0, E // CE, chunk_body, (z0,) * (2 * NRJ))
            for j in range(NRJ):
                fj, wj = fw[j], fw[NRJ + j]
                row = wid * NRJ + j
                pltpu.sync_copy(lsrc[j].at[pl.ds(0, FB)],
                                bs_h.at[pl.ds(pl.multiple_of(row * CAP + wj, FB), FB)])
                pltpu.sync_copy(ldst[j].at[pl.ds(0, FB)],
                                bd_h.at[pl.ds(pl.multiple_of(row * CAP + wj, FB), FB)])
                pltpu.sync_copy(lew[j].at[pl.ds(0, FB)],
                                be_h.at[pl.ds(pl.multiple_of(row * CAP + wj, FB), FB)])
                pltpu.sync_copy(lsrc[j].at[pl.ds(FB, 32)],
                                bs_h.at[pl.ds(pl.multiple_of(row * CAP + wj + FB, 8), 32)])
                pltpu.sync_copy(ldst[j].at[pl.ds(FB, 32)],
                                bd_h.at[pl.ds(pl.multiple_of(row * CAP + wj + FB, 8), 32)])
                pltpu.sync_copy(lew[j].at[pl.ds(FB, 32)],
                                be_h.at[pl.ds(pl.multiple_of(row * CAP + wj + FB, 8), 32)])
                cbuf[...] = jnp.zeros((16,), I32) + (wj + fj)
                pltpu.sync_copy(cbuf,
                                cnt_h.at[pl.ds(pl.multiple_of(row * 16, 16), 16)])
        else:
            def p1j(j, _):
                r = wid * NRJ + j

                @pl.when(r * RW < N)
                def _():
                    pltpu.sync_copy(cnt_h.at[pl.ds(pl.multiple_of(r * 16, 16), 16)],
                                    cbuf)
                    cnt = cbuf[...][0]
                    nch = jnp.right_shift(cnt + (CL - 1), 10)

                    def ch_body(k, _):
                        pltpu.sync_copy(bs_h.at[pl.ds(pl.multiple_of(r * CAP + k * CL, CL), CL)],
                                        c1)
                        pltpu.sync_copy(bd_h.at[pl.ds(pl.multiple_of(r * CAP + k * CL, CL), CL)],
                                        c2)
                        pltpu.sync_copy(be_h.at[pl.ds(pl.multiple_of(r * CAP + k * CL, CL), CL)],
                                        c3)

                        def g_body(g, _):
                            s16 = c1[pl.ds(g * 16, 16)]
                            d16 = c2[pl.ds(g * 16, 16)]
                            e16 = c3[pl.ds(g * 16, 16)]
                            gi = k * CL + g * 16 + lane
                            mm = gi < cnt
                            s16 = jnp.where(mm, s16, 0)
                            cidx = jnp.where(mm, d16 - base, 0)
                            sv = plsc.load_gather(ssrc_t, [s16])
                            dv = plsc.load_gather(sdst_l, [cidx])
                            ee = sv + dv + e16 * ae
                            ee = jnp.where(ee > 0, ee, ee * 0.2)
                            ex = jnp.exp(ee)
                            plsc.addupdate_scatter(sums_t, [lane, cidx], ex,
                                                   mask=mm)
                            return 0
                        lax.fori_loop(0, CL // 16, g_body, 0)
                        return 0
                    lax.fori_loop(0, nch, ch_body, 0)
                return 0
            lax.fori_loop(0, NRJ, p1j, 0)

        def fold_body(jj, _):
            acc = sums_t[0, pl.ds(jj * 16, 16)]
            for l in range(1, 16):
                acc = acc + sums_t[l, pl.ds(jj * 16, 16)]
            sloc_t[pl.ds(jj * 16, 16)] = acc
            return 0
        lax.fori_loop(0, SR // 16, fold_body, 0)

        def p2j(j, _):
            r = wid * NRJ + j

            @pl.when(r * RW < N)
            def _():
                pltpu.sync_copy(cnt_h.at[pl.ds(pl.multiple_of(r * 16, 16), 16)],
                                cbuf)
                cnt = cbuf[...][0]
                nch = jnp.right_shift(cnt + (CL - 1), 10)

                def zb(i, _):
                    for c in range(NV):
                        outblk[i, pl.ds(c * 16, 16)] = zv
                    return 0
                lax.fori_loop(0, RW, zb, 0)

                def ch_body(k, _):
                    pltpu.sync_copy(bs_h.at[pl.ds(pl.multiple_of(r * CAP + k * CL, CL), CL)],
                                    c1)
                    pltpu.sync_copy(bd_h.at[pl.ds(pl.multiple_of(r * CAP + k * CL, CL), CL)],
                                    c2)
                    pltpu.sync_copy(be_h.at[pl.ds(pl.multiple_of(r * CAP + k * CL, CL), CL)],
                                    c3)

                    @plsc.parallel_loop(0, CL // 16)
                    def _(g):
                        s16 = c1[pl.ds(g * 16, 16)]
                        d16 = c2[pl.ds(g * 16, 16)]
                        e16 = c3[pl.ds(g * 16, 16)]
                        gi = k * CL + g * 16 + lane
                        mm = gi < cnt
                        s16 = jnp.where(mm, s16, 0)
                        dloc = jnp.where(mm, jnp.bitwise_and(d16, RW - 1), 0)
                        cidx = dloc + j * RW
                        sv = plsc.load_gather(ssrc_t, [s16])
                        dv = plsc.load_gather(sdst_l, [cidx])
                        ee = sv + dv + e16 * ae
                        ee = jnp.where(ee > 0, ee, ee * 0.2)
                        ex = jnp.exp(ee)
                        den = plsc.load_gather(sloc_t, [cidx]) + 1e-9
                        al = jnp.where(mm, ex / den, 0.0)
                        ssan[pl.ds(g * 16, 16)] = s16
                        asan[pl.ds(g * 16, 16)] = al
                        dsan[pl.ds(g * 16, 16)] = dloc

                    def fire(g, st, sem):
                        pltpu.async_copy(
                            hw_h.at[ssan.at[pl.ds(g * G, G)]], st, sem)

                    def drain(st, sem):
                        pltpu.make_async_copy(
                            hw_h.at[ssan.at[pl.ds(0, G)]], st, sem).wait()

                    def accg(g, st):
                        @plsc.parallel_loop(0, G // 16)
                        def _(q):
                            av16 = asan[pl.ds(g * G + q * 16, 16)]
                            dl16 = dsan[pl.ds(g * G + q * 16, 16)]
                            for ii in range(16):
                                dl = dl16[ii]
                                av = av16[ii]
                                for half in range(2):
                                    vals = [av * st[q * 16 + ii,
                                                    pl.ds((half * 24 + c) * 16, 16)]
                                            for c in range(24)]
                                    for c in range(24):
                                        plsc.addupdate(
                                            outblk.at[dl, pl.ds((half * 24 + c) * 16, 16)],
                                            vals[c])

                    fire(0, stgA, semA)

                    def pipe(i, _):
                        fire(2 * i + 1, stgB, semB)
                        drain(stgA, semA)
                        accg(2 * i, stgA)

                        @pl.when(i < NGG // 2 - 1)
                        def _():
                            fire(2 * i + 2, stgA, semA)
                        drain(stgB, semB)
                        accg(2 * i + 1, stgB)
                        return 0
                    lax.fori_loop(0, NGG // 2, pipe, 0)
                    return 0
                lax.fori_loop(0, nch, ch_body, 0)

                pltpu.sync_copy(sn_h.at[pl.ds(pl.multiple_of(r * RW, RW), RW)],
                                snloc)

                @plsc.parallel_loop(0, RW // 16)
                def _(q):
                    sn16 = snloc[pl.ds(q * 16, 16)]
                    for ii in range(16):
                        sn = sn16[ii]
                        for half in range(2):
                            vals = [outblk[q * 16 + ii,
                                           pl.ds((half * 24 + c) * 16, 16)] * sn
                                    for c in range(24)]
                            for c in range(24):
                                v = vals[c]
                                outblk[q * 16 + ii, pl.ds((half * 24 + c) * 16, 16)] = (
                                    jnp.where(v > 0, v, v * 0.2))
                pltpu.sync_copy(outblk,
                                out_h.at[pl.ds(pl.multiple_of(r * RW, RW), RW)])
            return 0
        lax.fori_loop(0, NRJ, p2j, 0)

    return pl.kernel(body, out_type=out_type, mesh=mesh, scratch_types=scratch,
                     compiler_params=pltpu.CompilerParams(
                         needs_layout_passes=False,
                         use_tc_tiling_on_sc=False))


_sc_bucket = _make_sc(True)
_sc_reuse = _make_sc(False)


def _gat_mm(xp, W, a_s, a_d):
    D = W.shape[0]
    wp = jnp.zeros((FD, FD), F32).at[:D, :D].set(W.T)
    wa = jnp.zeros((FD, 128), F32).at[:D, 0].set(a_s).at[:D, 1].set(a_d)
    return _mm_gat(xp, wp, wa)


def _svecs(os_):
    ssrc = jnp.pad(os_[:N, 0], (0, NPAD - N))
    sdst = jnp.pad(os_[:N, 1], (0, NPAD - N))
    return ssrc, sdst


def kernel(features, edge_index, e_w, snorm_n, snorm_e, labels, maps_emb, eps,
           emb_W, emb_b,
           enc0_W, enc0_as, enc0_ad, enc0_ae,
           pri0_W, pri0_as, pri0_ad, pri0_ae,
           enc1_W, enc1_as, enc1_ad, enc1_ae,
           pri1_W, pri1_as, pri1_ad, pri1_ae,
           dec_W, dec_as, dec_ad,
           menc_W1, menc_b1, menc_Wmu, menc_bmu, menc_Wlv, menc_blv,
           mpri_W1, mpri_b1, mpri_Wmu, mpri_bmu, mpri_Wlv, mpri_blv,
           mdec_W0, mdec_b0, mdec_W1, mdec_b1):
    src = edge_index[0]
    dst = edge_index[1]
    ew = e_w[:, 0]
    sn = jnp.pad(snorm_n[:, 0], (0, NPAD - N))
    gt = labels

    h_emb = _mm(_pad2(features, MP, 128), _pad2(emb_W.T, 128, 128), emb_b)[:N, :128]

    x = _pad2(jnp.concatenate([maps_emb, h_emb, gt], axis=-1), MP, FD)
    hw, os_ = _gat_mm(x, enc0_W, enc0_as, enc0_ad)
    ssrc, sdst = _svecs(os_)
    ae = jnp.full((16,), enc0_ae[0], F32)
    out, bs, bd, be, cnts = _sc_bucket(src, dst, ew, ssrc, sdst, sn, ae, hw)

    x = _pad2(out[:N], MP, FD)
    hw, os_ = _gat_mm(x, enc1_W, enc1_as, enc1_ad)
    ssrc, sdst = _svecs(os_)
    ae = jnp.full((16,), enc1_ae[0], F32)
    out = _sc_reuse(bs, bd, be, cnts, ssrc, sdst, sn, ae, hw)[0]

    h = _pad2(jnp.concatenate([out[:N, :651], gt], axis=-1), MP, FD)
    hid = _mm(h, _pad2(menc_W1.T, FD, 384), menc_b1, act="lrelu")
    wmulv = (jnp.zeros((384, 256), F32)
             .at[:menc_Wmu.shape[1], 0:25].set(menc_Wmu.T)
             .at[:menc_Wlv.shape[1], 128:153].set(menc_Wlv.T))
    bmulv = (jnp.zeros((256,), F32).at[0:25].set(menc_bmu)
             .at[128:153].set(menc_blv))
    mu, lv = _mm_mulv(hid, wmulv, bmulv)
    z = _z_kernel(mu, lv, _pad2(eps, MP, 128))[:N, :25]

    x = _pad2(jnp.concatenate([maps_emb, h_emb, z], axis=-1), MP, FD)
    hw, os_ = _gat_mm(x, dec_W, dec_as, dec_ad)
    ssrc, sdst = _svecs(os_)
    out = _sc_reuse(bs, bd, be, cnts, ssrc, sdst, sn,
                    jnp.zeros((16,), F32), hw)[0]

    hd = jnp.concatenate([out[:N, :665], z], axis=-1)
    h0 = _mm(_pad2(hd, MP, FD), _pad2(mdec_W0.T, FD, FD), mdec_b0,
             act="lrelu")
    pred = _mm(h0, _pad2(mdec_W1.T, FD, 128), mdec_b1)[:N, :12]
    return pred


# speedup vs baseline: 1.0336x; 1.0336x over previous
import functools
import jax
import jax.numpy as jnp
from jax import lax
from jax.experimental import pallas as pl
from jax.experimental.pallas import tpu as pltpu
from jax.experimental.pallas import tpu_sc as plsc

N = 10000
E = 320000
MP = 10240
RW = 64
NR = 157
NRJ = 5
NWK = 32
NS = 16
SR = NRJ * RW
NPAD = MP
FD = 768
NV = FD // 16
CAP = E + 2048
CE = 800
CL = 1024
G = 16
NGG = CL // G
FB = 256
LB = FB + 32
BM = 512
F32 = jnp.float32
I32 = jnp.int32


def _pad2(x, m, n):
    M, Nc = x.shape
    return jnp.pad(x, ((0, m - M), (0, n - Nc)))


def _mm_body(x_ref, w_ref, b_ref, o_ref, *, act):
    acc = jnp.dot(x_ref[...], w_ref[...], preferred_element_type=F32)
    acc = acc + b_ref[...]
    if act == "lrelu":
        acc = jnp.where(acc > 0, acc, 0.2 * acc)
    o_ref[...] = acc


def _mm(x, w, b=None, act=None):
    M, K = x.shape
    K2, Nc = w.shape
    assert K == K2 and M % BM == 0, (x.shape, w.shape)
    if b is None:
        b = jnp.zeros((Nc,), F32)
    b2 = jnp.pad(b, (0, Nc - b.shape[0])).reshape(1, Nc)
    return pl.pallas_call(
        functools.partial(_mm_body, act=act),
        grid=(M // BM,),
        in_specs=[
            pl.BlockSpec((BM, K), lambda i: (i, 0)),
            pl.BlockSpec((K, Nc), lambda i: (0, 0)),
            pl.BlockSpec((1, Nc), lambda i: (0, 0)),
        ],
        out_specs=pl.BlockSpec((BM, Nc), lambda i: (i, 0)),
        out_shape=jax.ShapeDtypeStruct((M, Nc), F32),
    )(x, w, b2)


def _mm_gat_body(x_ref, w_ref, wa_ref, o_ref, os_ref):
    acc = jnp.dot(x_ref[...], w_ref[...], preferred_element_type=F32)
    o_ref[...] = acc
    os_ref[...] = jnp.dot(acc, wa_ref[...], preferred_element_type=F32)


def _mm_gat(x, w, wa):
    return pl.pallas_call(
        _mm_gat_body,
        grid=(MP // BM,),
        in_specs=[
            pl.BlockSpec((BM, FD), lambda i: (i, 0)),
            pl.BlockSpec((FD, FD), lambda i: (0, 0)),
            pl.BlockSpec((FD, 128), lambda i: (0, 0)),
        ],
        out_specs=[
            pl.BlockSpec((BM, FD), lambda i: (i, 0)),
            pl.BlockSpec((BM, 128), lambda i: (i, 0)),
        ],
        out_shape=[
            jax.ShapeDtypeStruct((MP, FD), F32),
            jax.ShapeDtypeStruct((MP, 128), F32),
        ],
    )(x, w, wa)


def _mulv_body(h_ref, w_ref, b_ref, mu_ref, lv_ref):
    acc = jnp.dot(h_ref[...], w_ref[...], preferred_element_type=F32) + b_ref[...]
    mu_ref[...] = acc[:, 0:128]
    lv_ref[...] = acc[:, 128:256]


def _mm_mulv(h, w, b):
    return pl.pallas_call(
        _mulv_body,
        grid=(MP // BM,),
        in_specs=[
            pl.BlockSpec((BM, 384), lambda i: (i, 0)),
            pl.BlockSpec((384, 256), lambda i: (0, 0)),
            pl.BlockSpec((1, 256), lambda i: (0, 0)),
        ],
        out_specs=[
            pl.BlockSpec((BM, 128), lambda i: (i, 0)),
            pl.BlockSpec((BM, 128), lambda i: (i, 0)),
        ],
        out_shape=[
            jax.ShapeDtypeStruct((MP, 128), F32),
            jax.ShapeDtypeStruct((MP, 128), F32),
        ],
    )(h, w, b.reshape(1, 256))


def _z_body(mu_ref, lv_ref, e_ref, z_ref):
    lv = 0.5 * lv_ref[...]
    std = jnp.where(lv > 0, lv, jnp.exp(lv) - 1.0) + (1.0 + 1e-5)
    z_ref[...] = mu_ref[...] + std * e_ref[...]


def _z_kernel(mu, lv, eps):
    return pl.pallas_call(
        _z_body,
        grid=(MP // BM,),
        in_specs=[pl.BlockSpec((BM, 128), lambda i: (i, 0))] * 3,
        out_specs=pl.BlockSpec((BM, 128), lambda i: (i, 0)),
        out_shape=jax.ShapeDtypeStruct((MP, 128), F32),
    )(mu, lv, eps)


def _make_sc(bucketize):
    mesh = plsc.VectorSubcoreMesh(core_axis_name="c", subcore_axis_name="s")
    out_type = [jax.ShapeDtypeStruct((NPAD, FD), F32)]
    if bucketize:
        out_type += [
            jax.ShapeDtypeStruct((NWK * NRJ * CAP,), I32),
            jax.ShapeDtypeStruct((NWK * NRJ * CAP,), I32),
            jax.ShapeDtypeStruct((NWK * NRJ * CAP,), F32),
            jax.ShapeDtypeStruct((NWK * NRJ * 16,), I32),
        ]
    scratch = [
        pltpu.VMEM((NPAD,), F32),
        pltpu.VMEM((SR,), F32),
        pltpu.VMEM((SR,), F32),
        pltpu.VMEM((16, SR), F32),
        pltpu.VMEM((CL,), I32),
        pltpu.VMEM((CL,), I32),
        pltpu.VMEM((CL,), F32),
        pltpu.VMEM((CL,), I32),
        pltpu.VMEM((CL,), F32),
        pltpu.VMEM((CL,), I32),
        pltpu.VMEM((RW, FD), F32),
        pltpu.VMEM((4, G, FD), F32),
        pltpu.VMEM((RW,), F32),
        pltpu.VMEM((16,), F32),
        pltpu.VMEM((16,), I32),
        pltpu.SemaphoreType.DMA((4,)),
    ]
    if bucketize:
        for _ in range(NRJ):
            scratch += [pltpu.VMEM((LB,), I32), pltpu.VMEM((LB,), I32),
                        pltpu.VMEM((LB,), F32)]

    def body(*refs):
        if bucketize:
            (src_h, dst_h, ew_h, ssrc_h, sdst_h, sn_h, ae_h, hw_h,
             out_h, bs_h, bd_h, be_h, cnt_h,
             ssrc_t, sdst_l, sloc_t, sums_t, c1, c2, c3, ssan, asan, dsan,
             outblk, stg, snloc, aev, cbuf, sem, *lbufs) = refs
            lsrc = [lbufs[3 * j] for j in range(NRJ)]
            ldst = [lbufs[3 * j + 1] for j in range(NRJ)]
            lew = [lbufs[3 * j + 2] for j in range(NRJ)]
        else:
            (bs_h, bd_h, be_h, cnt_h, ssrc_h, sdst_h, sn_h, ae_h, hw_h,
             out_h,
             ssrc_t, sdst_l, sloc_t, sums_t, c1, c2, c3, ssan, asan, dsan,
             outblk, stg, snloc, aev, cbuf, sem) = refs

        wid = lax.axis_index("c") * NS + lax.axis_index("s")
        base = wid * SR
        lane = lax.iota(I32, 16)
        zv = jnp.zeros((16,), F32)

        pltpu.sync_copy(ssrc_h, ssrc_t)
        pltpu.sync_copy(ae_h, aev)
        ae = aev[...][0]
        pltpu.sync_copy(sdst_h.at[pl.ds(pl.multiple_of(base, SR), SR)], sdst_l)

        def _zs(i, _):
            for c in range(SR // 16):
                sums_t[i, pl.ds(c * 16, 16)] = zv
            return 0
        lax.fori_loop(0, 16, _zs, 0)

        if bucketize:
            def chunk_body(ci, carry):
                pltpu.sync_copy(src_h.at[pl.ds(pl.multiple_of(ci * CE, 8), CE)],
                                c1.at[pl.ds(0, CE)])
                pltpu.sync_copy(dst_h.at[pl.ds(pl.multiple_of(ci * CE, 8), CE)],
                                c2.at[pl.ds(0, CE)])
                pltpu.sync_copy(ew_h.at[pl.ds(pl.multiple_of(ci * CE, 8), CE)],
                                c3.at[pl.ds(0, CE)])

                def g_body(g, cy):
                    s16 = c1[pl.ds(g * 16, 16)]
                    d16 = c2[pl.ds(g * 16, 16)]
                    e16 = c3[pl.ds(g * 16, 16)]
                    cidx = d16 - base
                    match = (cidx >= 0) & (cidx < SR)
                    cidx_s = jnp.where(match, cidx, 0)
                    which = jnp.right_shift(cidx_s, 6)
                    sv = plsc.load_gather(ssrc_t, [s16])
                    dv = plsc.load_gather(sdst_l, [cidx_s])
                    ee = sv + dv + e16 * ae
                    ee = jnp.where(ee > 0, ee, ee * 0.2)
                    ex = jnp.exp(ee)
                    plsc.addupdate_scatter(sums_t, [lane, cidx_s], ex,
                                           mask=match)
                    out = []
                    for j in range(NRJ):
                        fj, wj = cy[j], cy[NRJ + j]
                        mj = match & (which == j)
                        plsc.store_compressed(lsrc[j].at[pl.ds(fj, 16)], s16,
                                              mask=mj)
                        plsc.store_compressed(ldst[j].at[pl.ds(fj, 16)], d16,
                                              mask=mj)
                        plsc.store_compressed(lew[j].at[pl.ds(fj, 16)], e16,
                                              mask=mj)
                        fj = fj + plsc.all_reduce_population_count(mj)[0]
                        do = fj >= FB

                        @pl.when(do)
                        def _(j=j, wj=wj):
                            row = wid * NRJ + j
                            pltpu.sync_copy(
                                lsrc[j].at[pl.ds(0, FB)],
                                bs_h.at[pl.ds(pl.multiple_of(row * CAP + wj, FB), FB)])
                            pltpu.sync_copy(
                                ldst[j].at[pl.ds(0, FB)],
                                bd_h.at[pl.ds(pl.multiple_of(row * CAP + wj, FB), FB)])
                            pltpu.sync_copy(
                                lew[j].at[pl.ds(0, FB)],
                                be_h.at[pl.ds(pl.multiple_of(row * CAP + wj, FB), FB)])
                            t1 = lsrc[j][pl.ds(FB, 16)]
                            lsrc[j][pl.ds(0, 16)] = t1
                            t2 = ldst[j][pl.ds(FB, 16)]
                            ldst[j][pl.ds(0, 16)] = t2
                            t3 = lew[j][pl.ds(FB, 16)]
                            lew[j][pl.ds(0, 16)] = t3
                        out.append((jnp.where(do, fj - FB, fj),
                                    jnp.where(do, wj + FB, wj)))
                    return tuple([o[0] for o in out] + [o[1] for o in out])
                return lax.fori_loop(0, CE // 16, g_body, carry)

            z0 = jnp.zeros((), I32)
            fw = lax.fori_loop(0, E // CE, chunk_body, (z0,) * (2 * NRJ))
            for j in range(NRJ):
                fj, wj = fw[j], fw[NRJ + j]
                row = wid * NRJ + j
                pltpu.sync_copy(lsrc[j].at[pl.ds(0, FB)],
                                bs_h.at[pl.ds(pl.multiple_of(row * CAP + wj, FB), FB)])
                pltpu.sync_copy(ldst[j].at[pl.ds(0, FB)],
                                bd_h.at[pl.ds(pl.multiple_of(row * CAP + wj, FB), FB)])
                pltpu.sync_copy(lew[j].at[pl.ds(0, FB)],
                                be_h.at[pl.ds(pl.multiple_of(row * CAP + wj, FB), FB)])
                pltpu.sync_copy(lsrc[j].at[pl.ds(FB, 32)],
                                bs_h.at[pl.ds(pl.multiple_of(row * CAP + wj + FB, 8), 32)])
                pltpu.sync_copy(ldst[j].at[pl.ds(FB, 32)],
                                bd_h.at[pl.ds(pl.multiple_of(row * CAP + wj + FB, 8), 32)])
                pltpu.sync_copy(lew[j].at[pl.ds(FB, 32)],
                                be_h.at[pl.ds(pl.multiple_of(row * CAP + wj + FB, 8), 32)])
                cbuf[...] = jnp.zeros((16,), I32) + (wj + fj)
                pltpu.sync_copy(cbuf,
                                cnt_h.at[pl.ds(pl.multiple_of(row * 16, 16), 16)])
        else:
            def p1j(j, _):
                r = wid * NRJ + j
                _ = jax.named_scope

                @pl.when(r * RW < N)
                def _():
                    pltpu.sync_copy(cnt_h.at[pl.ds(pl.multiple_of(r * 16, 16), 16)],
                                    cbuf)
                    cnt = cbuf[...][0]
                    nch = jnp.right_shift(cnt + (CL - 1), 10)

                    def ch_body(k, _):
                        pltpu.sync_copy(bs_h.at[pl.ds(pl.multiple_of(r * CAP + k * CL, CL), CL)],
                                        c1)
                        pltpu.sync_copy(bd_h.at[pl.ds(pl.multiple_of(r * CAP + k * CL, CL), CL)],
                                        c2)
                        pltpu.sync_copy(be_h.at[pl.ds(pl.multiple_of(r * CAP + k * CL, CL), CL)],
                                        c3)

                        def g_body(g, _):
                            s16 = c1[pl.ds(g * 16, 16)]
                            d16 = c2[pl.ds(g * 16, 16)]
                            e16 = c3[pl.ds(g * 16, 16)]
                            gi = k * CL + g * 16 + lane
                            mm = gi < cnt
                            s16 = jnp.where(mm, s16, 0)
                            cidx = jnp.where(mm, d16 - base, 0)
                            sv = plsc.load_gather(ssrc_t, [s16])
                            dv = plsc.load_gather(sdst_l, [cidx])
                            ee = sv + dv + e16 * ae
                            ee = jnp.where(ee > 0, ee, ee * 0.2)
                            ex = jnp.exp(ee)
                            plsc.addupdate_scatter(sums_t, [lane, cidx], ex,
                                                   mask=mm)
                            return 0
                        lax.fori_loop(0, CL // 16, g_body, 0)
                        return 0
                    lax.fori_loop(0, nch, ch_body, 0)
                return 0
            lax.fori_loop(0, NRJ, p1j, 0)

        def fold_body(jj, _):
            acc = sums_t[0, pl.ds(jj * 16, 16)]
            for l in range(1, 16):
                acc = acc + sums_t[l, pl.ds(jj * 16, 16)]
            sloc_t[pl.ds(jj * 16, 16)] = acc
            return 0
        lax.fori_loop(0, SR // 16, fold_body, 0)

        def p2j(j, _):
            r = wid * NRJ + j

            @pl.when(r * RW < N)
            def _():
                pltpu.sync_copy(cnt_h.at[pl.ds(pl.multiple_of(r * 16, 16), 16)],
                                cbuf)
                cnt = cbuf[...][0]
                nch = jnp.right_shift(cnt + (CL - 1), 10)

                def zb(i, _):
                    for c in range(NV):
                        outblk[i, pl.ds(c * 16, 16)] = zv
                    return 0
                lax.fori_loop(0, RW, zb, 0)

                def ch_body(k, _):
                    pltpu.sync_copy(bs_h.at[pl.ds(pl.multiple_of(r * CAP + k * CL, CL), CL)],
                                    c1)
                    pltpu.sync_copy(bd_h.at[pl.ds(pl.multiple_of(r * CAP + k * CL, CL), CL)],
                                    c2)
                    pltpu.sync_copy(be_h.at[pl.ds(pl.multiple_of(r * CAP + k * CL, CL), CL)],
                                    c3)

                    @plsc.parallel_loop(0, CL // 16)
                    def _(g):
                        s16 = c1[pl.ds(g * 16, 16)]
                        d16 = c2[pl.ds(g * 16, 16)]
                        e16 = c3[pl.ds(g * 16, 16)]
                        gi = k * CL + g * 16 + lane
                        mm = gi < cnt
                        s16 = jnp.where(mm, s16, 0)
                        dloc = jnp.where(mm, jnp.bitwise_and(d16, RW - 1), 0)
                        cidx = dloc + j * RW
                        sv = plsc.load_gather(ssrc_t, [s16])
                        dv = plsc.load_gather(sdst_l, [cidx])
                        ee = sv + dv + e16 * ae
                        ee = jnp.where(ee > 0, ee, ee * 0.2)
                        ex = jnp.exp(ee)
                        den = plsc.load_gather(sloc_t, [cidx]) + 1e-9
                        al = jnp.where(mm, ex / den, 0.0)
                        ssan[pl.ds(g * 16, 16)] = s16
                        asan[pl.ds(g * 16, 16)] = al
                        dsan[pl.ds(g * 16, 16)] = dloc

                    def fire(g, b):
                        pltpu.async_copy(
                            hw_h.at[ssan.at[pl.ds(g * G, G)]], stg.at[b],
                            sem.at[b])

                    def drain(b):
                        pltpu.make_async_copy(
                            hw_h.at[ssan.at[pl.ds(0, G)]], stg.at[b],
                            sem.at[b]).wait()

                    for b in range(4):
                        fire(b, b)

                    def pipe(g, _):
                        b = jnp.bitwise_and(g, 3)
                        drain(b)

                        @plsc.parallel_loop(0, G // 16)
                        def _(q):
                            av16 = asan[pl.ds(g * G + q * 16, 16)]
                            dl16 = dsan[pl.ds(g * G + q * 16, 16)]
                            for ii in range(16):
                                dl = dl16[ii]
                                av = av16[ii]
                                for half in range(2):
                                    vals = [av * stg[b, q * 16 + ii,
                                                     pl.ds((half * 24 + c) * 16, 16)]
                                            for c in range(24)]
                                    for c in range(24):
                                        plsc.addupdate(
                                            outblk.at[dl, pl.ds((half * 24 + c) * 16, 16)],
                                            vals[c])

                        @pl.when(g + 4 < NGG)
                        def _():
                            fire(g + 4, b)
                        return 0
                    lax.fori_loop(0, NGG, pipe, 0)
                    return 0
                lax.fori_loop(0, nch, ch_body, 0)

                pltpu.sync_copy(sn_h.at[pl.ds(pl.multiple_of(r * RW, RW), RW)],
                                snloc)

                @plsc.parallel_loop(0, RW // 16)
                def _(q):
                    sn16 = snloc[pl.ds(q * 16, 16)]
                    for ii in range(16):
                        sn = sn16[ii]
                        for half in range(2):
                            vals = [outblk[q * 16 + ii,
                                           pl.ds((half * 24 + c) * 16, 16)] * sn
                                    for c in range(24)]
                            for c in range(24):
                                v = vals[c]
                                outblk[q * 16 + ii, pl.ds((half * 24 + c) * 16, 16)] = (
                                    jnp.where(v > 0, v, v * 0.2))
                pltpu.sync_copy(outblk,
                                out_h.at[pl.ds(pl.multiple_of(r * RW, RW), RW)])
            return 0
        lax.fori_loop(0, NRJ, p2j, 0)

    return pl.kernel(body, out_type=out_type, mesh=mesh, scratch_types=scratch,
                     compiler_params=pltpu.CompilerParams(
                         needs_layout_passes=False))


_sc_bucket = _make_sc(True)
_sc_reuse = _make_sc(False)


def _gat_mm(xp, W, a_s, a_d):
    D = W.shape[0]
    wp = jnp.zeros((FD, FD), F32).at[:D, :D].set(W.T)
    wa = jnp.zeros((FD, 128), F32).at[:D, 0].set(a_s).at[:D, 1].set(a_d)
    return _mm_gat(xp, wp, wa)


def _svecs(os_):
    ssrc = jnp.pad(os_[:N, 0], (0, NPAD - N))
    sdst = jnp.pad(os_[:N, 1], (0, NPAD - N))
    return ssrc, sdst


def kernel(features, edge_index, e_w, snorm_n, snorm_e, labels, maps_emb, eps,
           emb_W, emb_b,
           enc0_W, enc0_as, enc0_ad, enc0_ae,
           pri0_W, pri0_as, pri0_ad, pri0_ae,
           enc1_W, enc1_as, enc1_ad, enc1_ae,
           pri1_W, pri1_as, pri1_ad, pri1_ae,
           dec_W, dec_as, dec_ad,
           menc_W1, menc_b1, menc_Wmu, menc_bmu, menc_Wlv, menc_blv,
           mpri_W1, mpri_b1, mpri_Wmu, mpri_bmu, mpri_Wlv, mpri_blv,
           mdec_W0, mdec_b0, mdec_W1, mdec_b1):
    src = edge_index[0]
    dst = edge_index[1]
    ew = e_w[:, 0]
    sn = jnp.pad(snorm_n[:, 0], (0, NPAD - N))
    gt = labels

    h_emb = _mm(_pad2(features, MP, 128), _pad2(emb_W.T, 128, 128), emb_b)[:N, :128]

    x = _pad2(jnp.concatenate([maps_emb, h_emb, gt], axis=-1), MP, FD)
    hw, os_ = _gat_mm(x, enc0_W, enc0_as, enc0_ad)
    ssrc, sdst = _svecs(os_)
    ae = jnp.full((16,), enc0_ae[0], F32)
    out, bs, bd, be, cnts = _sc_bucket(src, dst, ew, ssrc, sdst, sn, ae, hw)

    x = _pad2(out[:N], MP, FD)
    hw, os_ = _gat_mm(x, enc1_W, enc1_as, enc1_ad)
    ssrc, sdst = _svecs(os_)
    ae = jnp.full((16,), enc1_ae[0], F32)
    out = _sc_reuse(bs, bd, be, cnts, ssrc, sdst, sn, ae, hw)[0]

    h = _pad2(jnp.concatenate([out[:N, :651], gt], axis=-1), MP, FD)
    hid = _mm(h, _pad2(menc_W1.T, FD, 384), menc_b1, act="lrelu")
    wmulv = (jnp.zeros((384, 256), F32)
             .at[:menc_Wmu.shape[1], 0:25].set(menc_Wmu.T)
             .at[:menc_Wlv.shape[1], 128:153].set(menc_Wlv.T))
    bmulv = (jnp.zeros((256,), F32).at[0:25].set(menc_bmu)
             .at[128:153].set(menc_blv))
    mu, lv = _mm_mulv(hid, wmulv, bmulv)
    z = _z_kernel(mu, lv, _pad2(eps, MP, 128))[:N, :25]

    x = _pad2(jnp.concatenate([maps_emb, h_emb, z], axis=-1), MP, FD)
    hw, os_ = _gat_mm(x, dec_W, dec_as, dec_ad)
    ssrc, sdst = _svecs(os_)
    out = _sc_reuse(bs, bd, be, cnts, ssrc, sdst, sn,
                    jnp.zeros((16,), F32), hw)[0]

    hd = jnp.concatenate([out[:N, :665], z], axis=-1)
    h0 = _mm(_pad2(hd, MP, FD), _pad2(mdec_W0.T, FD, FD), mdec_b0,
             act="lrelu")
    pred = _mm(h0, _pad2(mdec_W1.T, FD, 128), mdec_b1)[:N, :12]
    return pred


# speedup vs baseline: 1.0364x; 1.0028x over previous
import functools
import jax
import jax.numpy as jnp
from jax import lax
from jax.experimental import pallas as pl
from jax.experimental.pallas import tpu as pltpu
from jax.experimental.pallas import tpu_sc as plsc

N = 10000
E = 320000
MP = 10240
RW = 64
NR = 157
NRJ = 5
NWK = 32
NS = 16
SR = NRJ * RW
NPAD = MP
FD = 768
NV = FD // 16
CAP = E + 2048
CE = 800
CL = 1024
G = 16
NGG = CL // G
FB = 256
LB = FB + 32
BM = 512
F32 = jnp.float32
I32 = jnp.int32


def _pad2(x, m, n):
    M, Nc = x.shape
    return jnp.pad(x, ((0, m - M), (0, n - Nc)))


def _mm_body(x_ref, w_ref, b_ref, o_ref, *, act):
    acc = jnp.dot(x_ref[...], w_ref[...], preferred_element_type=F32)
    acc = acc + b_ref[...]
    if act == "lrelu":
        acc = jnp.where(acc > 0, acc, 0.2 * acc)
    o_ref[...] = acc


def _mm(x, w, b=None, act=None):
    M, K = x.shape
    K2, Nc = w.shape
    assert K == K2 and M % BM == 0, (x.shape, w.shape)
    if b is None:
        b = jnp.zeros((Nc,), F32)
    b2 = jnp.pad(b, (0, Nc - b.shape[0])).reshape(1, Nc)
    return pl.pallas_call(
        functools.partial(_mm_body, act=act),
        grid=(M // BM,),
        in_specs=[
            pl.BlockSpec((BM, K), lambda i: (i, 0)),
            pl.BlockSpec((K, Nc), lambda i: (0, 0)),
            pl.BlockSpec((1, Nc), lambda i: (0, 0)),
        ],
        out_specs=pl.BlockSpec((BM, Nc), lambda i: (i, 0)),
        out_shape=jax.ShapeDtypeStruct((M, Nc), F32),
    )(x, w, b2)


def _mm_gat_body(x_ref, w_ref, wa_ref, o_ref, os_ref):
    acc = jnp.dot(x_ref[...], w_ref[...], preferred_element_type=F32)
    o_ref[...] = acc
    os_ref[...] = jnp.dot(acc, wa_ref[...], preferred_element_type=F32)


def _mm_gat(x, w, wa):
    return pl.pallas_call(
        _mm_gat_body,
        grid=(MP // BM,),
        in_specs=[
            pl.BlockSpec((BM, FD), lambda i: (i, 0)),
            pl.BlockSpec((FD, FD), lambda i: (0, 0)),
            pl.BlockSpec((FD, 128), lambda i: (0, 0)),
        ],
        out_specs=[
            pl.BlockSpec((BM, FD), lambda i: (i, 0)),
            pl.BlockSpec((BM, 128), lambda i: (i, 0)),
        ],
        out_shape=[
            jax.ShapeDtypeStruct((MP, FD), F32),
            jax.ShapeDtypeStruct((MP, 128), F32),
        ],
    )(x, w, wa)


def _mulv_body(h_ref, w_ref, b_ref, mu_ref, lv_ref):
    acc = jnp.dot(h_ref[...], w_ref[...], preferred_element_type=F32) + b_ref[...]
    mu_ref[...] = acc[:, 0:128]
    lv_ref[...] = acc[:, 128:256]


def _mm_mulv(h, w, b):
    return pl.pallas_call(
        _mulv_body,
        grid=(MP // BM,),
        in_specs=[
            pl.BlockSpec((BM, 384), lambda i: (i, 0)),
            pl.BlockSpec((384, 256), lambda i: (0, 0)),
            pl.BlockSpec((1, 256), lambda i: (0, 0)),
        ],
        out_specs=[
            pl.BlockSpec((BM, 128), lambda i: (i, 0)),
            pl.BlockSpec((BM, 128), lambda i: (i, 0)),
        ],
        out_shape=[
            jax.ShapeDtypeStruct((MP, 128), F32),
            jax.ShapeDtypeStruct((MP, 128), F32),
        ],
    )(h, w, b.reshape(1, 256))


def _z_body(mu_ref, lv_ref, e_ref, z_ref):
    lv = 0.5 * lv_ref[...]
    std = jnp.where(lv > 0, lv, jnp.exp(lv) - 1.0) + (1.0 + 1e-5)
    z_ref[...] = mu_ref[...] + std * e_ref[...]


def _z_kernel(mu, lv, eps):
    return pl.pallas_call(
        _z_body,
        grid=(MP // BM,),
        in_specs=[pl.BlockSpec((BM, 128), lambda i: (i, 0))] * 3,
        out_specs=pl.BlockSpec((BM, 128), lambda i: (i, 0)),
        out_shape=jax.ShapeDtypeStruct((MP, 128), F32),
    )(mu, lv, eps)


def _make_sc(bucketize):
    mesh = plsc.VectorSubcoreMesh(core_axis_name="c", subcore_axis_name="s")
    out_type = [jax.ShapeDtypeStruct((NPAD, FD), F32)]
    if bucketize:
        out_type += [
            jax.ShapeDtypeStruct((NWK * NRJ * CAP,), I32),
            jax.ShapeDtypeStruct((NWK * NRJ * CAP,), I32),
            jax.ShapeDtypeStruct((NWK * NRJ * CAP,), F32),
            jax.ShapeDtypeStruct((NWK * NRJ * 16,), I32),
        ]
    scratch = [
        pltpu.VMEM((NPAD,), F32),
        pltpu.VMEM((SR,), F32),
        pltpu.VMEM((SR,), F32),
        pltpu.VMEM((16, SR), F32),
        pltpu.VMEM((2 * CL,), I32),
        pltpu.VMEM((2 * CL,), I32),
        pltpu.VMEM((2 * CL,), F32),
        pltpu.VMEM((CL,), I32),
        pltpu.VMEM((CL,), F32),
        pltpu.VMEM((CL,), I32),
        pltpu.VMEM((RW, FD), F32),
        pltpu.VMEM((2, G, FD), F32),
        pltpu.VMEM((RW,), F32),
        pltpu.VMEM((16,), F32),
        pltpu.VMEM((16,), I32),
        pltpu.SemaphoreType.DMA((2,)),
        pltpu.SemaphoreType.DMA,
    ]
    if bucketize:
        for _ in range(NRJ):
            scratch += [pltpu.VMEM((LB,), I32), pltpu.VMEM((LB,), I32),
                        pltpu.VMEM((LB,), F32)]

    def body(*refs):
        if bucketize:
            (src_h, dst_h, ew_h, ssrc_h, sdst_h, sn_h, ae_h, hw_h,
             out_h, bs_h, bd_h, be_h, cnt_h,
             ssrc_t, sdst_l, sloc_t, sums_t, c1, c2, c3, ssan, asan, dsan,
             outblk, stg, snloc, aev, cbuf, sem, csem, *lbufs) = refs
            lsrc = [lbufs[3 * j] for j in range(NRJ)]
            ldst = [lbufs[3 * j + 1] for j in range(NRJ)]
            lew = [lbufs[3 * j + 2] for j in range(NRJ)]
        else:
            (bs_h, bd_h, be_h, cnt_h, ssrc_h, sdst_h, sn_h, ae_h, hw_h,
             out_h,
             ssrc_t, sdst_l, sloc_t, sums_t, c1, c2, c3, ssan, asan, dsan,
             outblk, stg, snloc, aev, cbuf, sem, csem) = refs

        wid = lax.axis_index("c") * NS + lax.axis_index("s")
        base = wid * SR
        lane = lax.iota(I32, 16)
        zv = jnp.zeros((16,), F32)

        pltpu.sync_copy(ssrc_h, ssrc_t)
        pltpu.sync_copy(ae_h, aev)
        ae = aev[...][0]
        pltpu.sync_copy(sdst_h.at[pl.ds(pl.multiple_of(base, SR), SR)], sdst_l)

        def _zs(i, _):
            for c in range(SR // 16):
                sums_t[i, pl.ds(c * 16, 16)] = zv
            return 0
        lax.fori_loop(0, 16, _zs, 0)

        if bucketize:
            def chunk_body(ci, carry):
                pltpu.sync_copy(src_h.at[pl.ds(pl.multiple_of(ci * CE, 8), CE)],
                                c1.at[pl.ds(0, CE)])
                pltpu.sync_copy(dst_h.at[pl.ds(pl.multiple_of(ci * CE, 8), CE)],
                                c2.at[pl.ds(0, CE)])
                pltpu.sync_copy(ew_h.at[pl.ds(pl.multiple_of(ci * CE, 8), CE)],
                                c3.at[pl.ds(0, CE)])

                def g_body(g, cy):
                    s16 = c1[pl.ds(g * 16, 16)]
                    d16 = c2[pl.ds(g * 16, 16)]
                    e16 = c3[pl.ds(g * 16, 16)]
                    cidx = d16 - base
                    match = (cidx >= 0) & (cidx < SR)
                    cidx_s = jnp.where(match, cidx, 0)
                    which = jnp.right_shift(cidx_s, 6)
                    sv = plsc.load_gather(ssrc_t, [s16])
                    dv = plsc.load_gather(sdst_l, [cidx_s])
                    ee = sv + dv + e16 * ae
                    ee = jnp.where(ee > 0, ee, ee * 0.2)
                    ex = jnp.exp(ee)
                    plsc.addupdate_scatter(sums_t, [lane, cidx_s], ex,
                                           mask=match)
                    out = []
                    for j in range(NRJ):
                        fj, wj = cy[j], cy[NRJ + j]
                        mj = match & (which == j)
                        plsc.store_compressed(lsrc[j].at[pl.ds(fj, 16)], s16,
                                              mask=mj)
                        plsc.store_compressed(ldst[j].at[pl.ds(fj, 16)], d16,
                                              mask=mj)
                        plsc.store_compressed(lew[j].at[pl.ds(fj, 16)], e16,
                                              mask=mj)
                        fj = fj + plsc.all_reduce_population_count(mj)[0]
                        do = fj >= FB

                        @pl.when(do)
                        def _(j=j, wj=wj):
                            row = wid * NRJ + j
                            pltpu.sync_copy(
                                lsrc[j].at[pl.ds(0, FB)],
                                bs_h.at[pl.ds(pl.multiple_of(row * CAP + wj, FB), FB)])
                            pltpu.sync_copy(
                                ldst[j].at[pl.ds(0, FB)],
                                bd_h.at[pl.ds(pl.multiple_of(row * CAP + wj, FB), FB)])
                            pltpu.sync_copy(
                                lew[j].at[pl.ds(0, FB)],
                                be_h.at[pl.ds(pl.multiple_of(row * CAP + wj, FB), FB)])
                            t1 = lsrc[j][pl.ds(FB, 16)]
                            lsrc[j][pl.ds(0, 16)] = t1
                            t2 = ldst[j][pl.ds(FB, 16)]
                            ldst[j][pl.ds(0, 16)] = t2
                            t3 = lew[j][pl.ds(FB, 16)]
                            lew[j][pl.ds(0, 16)] = t3
                        out.append((jnp.where(do, fj - FB, fj),
                                    jnp.where(do, wj + FB, wj)))
                    return tuple([o[0] for o in out] + [o[1] for o in out])
                return lax.fori_loop(0, CE // 16, g_body, carry)

            z0 = jnp.zeros((), I32)
            fw = lax.fori_loop(0, E // CE, chunk_body, (z0,) * (2 * NRJ))
            for j in range(NRJ):
                fj, wj = fw[j], fw[NRJ + j]
                row = wid * NRJ + j
                pltpu.sync_copy(lsrc[j].at[pl.ds(0, FB)],
                                bs_h.at[pl.ds(pl.multiple_of(row * CAP + wj, FB), FB)])
                pltpu.sync_copy(ldst[j].at[pl.ds(0, FB)],
                                bd_h.at[pl.ds(pl.multiple_of(row * CAP + wj, FB), FB)])
                pltpu.sync_copy(lew[j].at[pl.ds(0, FB)],
                                be_h.at[pl.ds(pl.multiple_of(row * CAP + wj, FB), FB)])
                pltpu.sync_copy(lsrc[j].at[pl.ds(FB, 32)],
                                bs_h.at[pl.ds(pl.multiple_of(row * CAP + wj + FB, 8), 32)])
                pltpu.sync_copy(ldst[j].at[pl.ds(FB, 32)],
                                bd_h.at[pl.ds(pl.multiple_of(row * CAP + wj + FB, 8), 32)])
                pltpu.sync_copy(lew[j].at[pl.ds(FB, 32)],
                                be_h.at[pl.ds(pl.multiple_of(row * CAP + wj + FB, 8), 32)])
                cbuf[...] = jnp.zeros((16,), I32) + (wj + fj)
                pltpu.sync_copy(cbuf,
                                cnt_h.at[pl.ds(pl.multiple_of(row * 16, 16), 16)])
        else:
            def p1j(j, _):
                r = wid * NRJ + j
                _ = jax.named_scope

                @pl.when(r * RW < N)
                def _():
                    pltpu.sync_copy(cnt_h.at[pl.ds(pl.multiple_of(r * 16, 16), 16)],
                                    cbuf)
                    cnt = cbuf[...][0]
                    nch = jnp.right_shift(cnt + (CL - 1), 10)

                    def ch_body(k, _):
                        pltpu.sync_copy(bs_h.at[pl.ds(pl.multiple_of(r * CAP + k * CL, CL), CL)],
                                        c1.at[pl.ds(0, CL)])
                        pltpu.sync_copy(bd_h.at[pl.ds(pl.multiple_of(r * CAP + k * CL, CL), CL)],
                                        c2.at[pl.ds(0, CL)])
                        pltpu.sync_copy(be_h.at[pl.ds(pl.multiple_of(r * CAP + k * CL, CL), CL)],
                                        c3.at[pl.ds(0, CL)])

                        def g_body(g, _):
                            s16 = c1[pl.ds(g * 16, 16)]
                            d16 = c2[pl.ds(g * 16, 16)]
                            e16 = c3[pl.ds(g * 16, 16)]
                            gi = k * CL + g * 16 + lane
                            mm = gi < cnt
                            s16 = jnp.where(mm, s16, 0)
                            cidx = jnp.where(mm, d16 - base, 0)
                            sv = plsc.load_gather(ssrc_t, [s16])
                            dv = plsc.load_gather(sdst_l, [cidx])
                            ee = sv + dv + e16 * ae
                            ee = jnp.where(ee > 0, ee, ee * 0.2)
                            ex = jnp.exp(ee)
                            plsc.addupdate_scatter(sums_t, [lane, cidx], ex,
                                                   mask=mm)
                            return 0
                        lax.fori_loop(0, CL // 16, g_body, 0)
                        return 0
                    lax.fori_loop(0, nch, ch_body, 0)
                return 0
            lax.fori_loop(0, NRJ, p1j, 0)

        def fold_body(jj, _):
            acc = sums_t[0, pl.ds(jj * 16, 16)]
            for l in range(1, 16):
                acc = acc + sums_t[l, pl.ds(jj * 16, 16)]
            sloc_t[pl.ds(jj * 16, 16)] = acc
            return 0
        lax.fori_loop(0, SR // 16, fold_body, 0)

        def p2j(j, _):
            r = wid * NRJ + j

            @pl.when(r * RW < N)
            def _():
                pltpu.sync_copy(cnt_h.at[pl.ds(pl.multiple_of(r * 16, 16), 16)],
                                cbuf)
                cnt = cbuf[...][0]
                nch = jnp.right_shift(cnt + (CL - 1), 10)

                def zb(i, _):
                    for c in range(NV):
                        outblk[i, pl.ds(c * 16, 16)] = zv
                    return 0
                lax.fori_loop(0, RW, zb, 0)

                def cfetch(k, p):
                    pltpu.async_copy(bs_h.at[pl.ds(pl.multiple_of(r * CAP + k * CL, CL), CL)],
                                     c1.at[pl.ds(pl.multiple_of(p * CL, CL), CL)], csem)
                    pltpu.async_copy(bd_h.at[pl.ds(pl.multiple_of(r * CAP + k * CL, CL), CL)],
                                     c2.at[pl.ds(pl.multiple_of(p * CL, CL), CL)], csem)
                    pltpu.async_copy(be_h.at[pl.ds(pl.multiple_of(r * CAP + k * CL, CL), CL)],
                                     c3.at[pl.ds(pl.multiple_of(p * CL, CL), CL)], csem)

                def cdrain(p):
                    pltpu.make_async_copy(bs_h.at[pl.ds(0, CL)], c1.at[pl.ds(pl.multiple_of(p * CL, CL), CL)],
                                          csem).wait()
                    pltpu.make_async_copy(bd_h.at[pl.ds(0, CL)], c2.at[pl.ds(pl.multiple_of(p * CL, CL), CL)],
                                          csem).wait()
                    pltpu.make_async_copy(be_h.at[pl.ds(0, CL)], c3.at[pl.ds(pl.multiple_of(p * CL, CL), CL)],
                                          csem).wait()

                @pl.when(nch > 0)
                def _():
                    cfetch(0, 0)

                def ch_body(k, _):
                    p = jnp.bitwise_and(k, 1)
                    cdrain(p)

                    @pl.when(k + 1 < nch)
                    def _():
                        cfetch(k + 1, 1 - p)

                    @plsc.parallel_loop(0, CL // 16)
                    def _(g):
                        s16 = c1[pl.ds(p * CL + g * 16, 16)]
                        d16 = c2[pl.ds(p * CL + g * 16, 16)]
                        e16 = c3[pl.ds(p * CL + g * 16, 16)]
                        gi = k * CL + g * 16 + lane
                        mm = gi < cnt
                        s16 = jnp.where(mm, s16, 0)
                        dloc = jnp.where(mm, jnp.bitwise_and(d16, RW - 1), 0)
                        cidx = dloc + j * RW
                        sv = plsc.load_gather(ssrc_t, [s16])
                        dv = plsc.load_gather(sdst_l, [cidx])
                        ee = sv + dv + e16 * ae
                        ee = jnp.where(ee > 0, ee, ee * 0.2)
                        ex = jnp.exp(ee)
                        den = plsc.load_gather(sloc_t, [cidx]) + 1e-9
                        al = jnp.where(mm, ex / den, 0.0)
                        ssan[pl.ds(g * 16, 16)] = s16
                        asan[pl.ds(g * 16, 16)] = al
                        dsan[pl.ds(g * 16, 16)] = dloc

                    def fire(g, b):
                        pltpu.async_copy(
                            hw_h.at[ssan.at[pl.ds(g * G, G)]], stg.at[b],
                            sem.at[b])

                    def drain(b):
                        pltpu.make_async_copy(
                            hw_h.at[ssan.at[pl.ds(0, G)]], stg.at[b],
                            sem.at[b]).wait()

                    for b in range(2):
                        fire(b, b)

                    def pipe(g, _):
                        b = jnp.bitwise_and(g, 1)
                        drain(b)

                        @plsc.parallel_loop(0, G // 16)
                        def _(q):
                            av16 = asan[pl.ds(g * G + q * 16, 16)]
                            dl16 = dsan[pl.ds(g * G + q * 16, 16)]
                            for ii in range(16):
                                dl = dl16[ii]
                                av = av16[ii]
                                for half in range(2):
                                    vals = [av * stg[b, q * 16 + ii,
                                                     pl.ds((half * 24 + c) * 16, 16)]
                                            for c in range(24)]
                                    for c in range(24):
                                        plsc.addupdate(
                                            outblk.at[dl, pl.ds((half * 24 + c) * 16, 16)],
                                            vals[c])

                        @pl.when(g + 2 < NGG)
                        def _():
                            fire(g + 2, b)
                        return 0
                    lax.fori_loop(0, NGG, pipe, 0)
                    return 0
                lax.fori_loop(0, nch, ch_body, 0)

                pltpu.sync_copy(sn_h.at[pl.ds(pl.multiple_of(r * RW, RW), RW)],
                                snloc)

                @plsc.parallel_loop(0, RW // 16)
                def _(q):
                    sn16 = snloc[pl.ds(q * 16, 16)]
                    for ii in range(16):
                        sn = sn16[ii]
                        for half in range(2):
                            vals = [outblk[q * 16 + ii,
                                           pl.ds((half * 24 + c) * 16, 16)] * sn
                                    for c in range(24)]
                            for c in range(24):
                                v = vals[c]
                                outblk[q * 16 + ii, pl.ds((half * 24 + c) * 16, 16)] = (
                                    jnp.where(v > 0, v, v * 0.2))
                pltpu.sync_copy(outblk,
                                out_h.at[pl.ds(pl.multiple_of(r * RW, RW), RW)])
            return 0
        lax.fori_loop(0, NRJ, p2j, 0)

    return pl.kernel(body, out_type=out_type, mesh=mesh, scratch_types=scratch,
                     compiler_params=pltpu.CompilerParams(
                         needs_layout_passes=False))


_sc_bucket = _make_sc(True)
_sc_reuse = _make_sc(False)


def _gat_mm(xp, W, a_s, a_d):
    D = W.shape[0]
    wp = jnp.zeros((FD, FD), F32).at[:D, :D].set(W.T)
    wa = jnp.zeros((FD, 128), F32).at[:D, 0].set(a_s).at[:D, 1].set(a_d)
    return _mm_gat(xp, wp, wa)


def _svecs(os_):
    ssrc = jnp.pad(os_[:N, 0], (0, NPAD - N))
    sdst = jnp.pad(os_[:N, 1], (0, NPAD - N))
    return ssrc, sdst


def kernel(features, edge_index, e_w, snorm_n, snorm_e, labels, maps_emb, eps,
           emb_W, emb_b,
           enc0_W, enc0_as, enc0_ad, enc0_ae,
           pri0_W, pri0_as, pri0_ad, pri0_ae,
           enc1_W, enc1_as, enc1_ad, enc1_ae,
           pri1_W, pri1_as, pri1_ad, pri1_ae,
           dec_W, dec_as, dec_ad,
           menc_W1, menc_b1, menc_Wmu, menc_bmu, menc_Wlv, menc_blv,
           mpri_W1, mpri_b1, mpri_Wmu, mpri_bmu, mpri_Wlv, mpri_blv,
           mdec_W0, mdec_b0, mdec_W1, mdec_b1):
    src = edge_index[0]
    dst = edge_index[1]
    ew = e_w[:, 0]
    sn = jnp.pad(snorm_n[:, 0], (0, NPAD - N))
    gt = labels

    h_emb = _mm(_pad2(features, MP, 128), _pad2(emb_W.T, 128, 128), emb_b)[:N, :128]

    x = _pad2(jnp.concatenate([maps_emb, h_emb, gt], axis=-1), MP, FD)
    hw, os_ = _gat_mm(x, enc0_W, enc0_as, enc0_ad)
    ssrc, sdst = _svecs(os_)
    ae = jnp.full((16,), enc0_ae[0], F32)
    out, bs, bd, be, cnts = _sc_bucket(src, dst, ew, ssrc, sdst, sn, ae, hw)

    x = _pad2(out[:N], MP, FD)
    hw, os_ = _gat_mm(x, enc1_W, enc1_as, enc1_ad)
    ssrc, sdst = _svecs(os_)
    ae = jnp.full((16,), enc1_ae[0], F32)
    out = _sc_reuse(bs, bd, be, cnts, ssrc, sdst, sn, ae, hw)[0]

    h = _pad2(jnp.concatenate([out[:N, :651], gt], axis=-1), MP, FD)
    hid = _mm(h, _pad2(menc_W1.T, FD, 384), menc_b1, act="lrelu")
    wmulv = (jnp.zeros((384, 256), F32)
             .at[:menc_Wmu.shape[1], 0:25].set(menc_Wmu.T)
             .at[:menc_Wlv.shape[1], 128:153].set(menc_Wlv.T))
    bmulv = (jnp.zeros((256,), F32).at[0:25].set(menc_bmu)
             .at[128:153].set(menc_blv))
    mu, lv = _mm_mulv(hid, wmulv, bmulv)
    z = _z_kernel(mu, lv, _pad2(eps, MP, 128))[:N, :25]

    x = _pad2(jnp.concatenate([maps_emb, h_emb, z], axis=-1), MP, FD)
    hw, os_ = _gat_mm(x, dec_W, dec_as, dec_ad)
    ssrc, sdst = _svecs(os_)
    out = _sc_reuse(bs, bd, be, cnts, ssrc, sdst, sn,
                    jnp.zeros((16,), F32), hw)[0]

    hd = jnp.concatenate([out[:N, :665], z], axis=-1)
    h0 = _mm(_pad2(hd, MP, FD), _pad2(mdec_W0.T, FD, FD), mdec_b0,
             act="lrelu")
    pred = _mm(h0, _pad2(mdec_W1.T, FD, 128), mdec_b1)[:N, :12]
    return pred


# speedup vs baseline: 1.0853x; 1.0471x over previous
import functools
import jax
import jax.numpy as jnp
from jax import lax
from jax.experimental import pallas as pl
from jax.experimental.pallas import tpu as pltpu
from jax.experimental.pallas import tpu_sc as plsc

N = 10000
E = 320000
MP = 10240
RW = 64
NR = 157
NRJ = 5
NWK = 32
NS = 16
SR = NRJ * RW
NPAD = MP
FD = 768
NV = FD // 16
CAP = E + 2048
CE = 800
CL = 1024
G = 16
NGG = CL // G
FB = 256
LB = FB + 32
BM = 512
F32 = jnp.float32
I32 = jnp.int32


def _pad2(x, m, n):
    M, Nc = x.shape
    return jnp.pad(x, ((0, m - M), (0, n - Nc)))


def _mm_body(x_ref, w_ref, b_ref, o_ref, *, act):
    acc = jnp.dot(x_ref[...], w_ref[...], preferred_element_type=F32)
    acc = acc + b_ref[...]
    if act == "lrelu":
        acc = jnp.where(acc > 0, acc, 0.2 * acc)
    o_ref[...] = acc


def _mm(x, w, b=None, act=None):
    M, K = x.shape
    K2, Nc = w.shape
    assert K == K2 and M % BM == 0, (x.shape, w.shape)
    if b is None:
        b = jnp.zeros((Nc,), F32)
    b2 = jnp.pad(b, (0, Nc - b.shape[0])).reshape(1, Nc)
    return pl.pallas_call(
        functools.partial(_mm_body, act=act),
        grid=(M // BM,),
        in_specs=[
            pl.BlockSpec((BM, K), lambda i: (i, 0)),
            pl.BlockSpec((K, Nc), lambda i: (0, 0)),
            pl.BlockSpec((1, Nc), lambda i: (0, 0)),
        ],
        out_specs=pl.BlockSpec((BM, Nc), lambda i: (i, 0)),
        out_shape=jax.ShapeDtypeStruct((M, Nc), F32),
    )(x, w, b2)


def _mm_gat_body(x_ref, w_ref, wa_ref, o_ref, os_ref):
    acc = jnp.dot(x_ref[...], w_ref[...], preferred_element_type=F32)
    o_ref[...] = acc.astype(jnp.bfloat16)
    os_ref[...] = jnp.dot(acc, wa_ref[...], preferred_element_type=F32)


def _mm_gat(x, w, wa):
    return pl.pallas_call(
        _mm_gat_body,
        grid=(MP // BM,),
        in_specs=[
            pl.BlockSpec((BM, FD), lambda i: (i, 0)),
            pl.BlockSpec((FD, FD), lambda i: (0, 0)),
            pl.BlockSpec((FD, 128), lambda i: (0, 0)),
        ],
        out_specs=[
            pl.BlockSpec((BM, FD), lambda i: (i, 0)),
            pl.BlockSpec((BM, 128), lambda i: (i, 0)),
        ],
        out_shape=[
            jax.ShapeDtypeStruct((MP, FD), jnp.bfloat16),
            jax.ShapeDtypeStruct((MP, 128), F32),
        ],
    )(x, w, wa)


def _mulv_body(h_ref, w_ref, b_ref, mu_ref, lv_ref):
    acc = jnp.dot(h_ref[...], w_ref[...], preferred_element_type=F32) + b_ref[...]
    mu_ref[...] = acc[:, 0:128]
    lv_ref[...] = acc[:, 128:256]


def _mm_mulv(h, w, b):
    return pl.pallas_call(
        _mulv_body,
        grid=(MP // BM,),
        in_specs=[
            pl.BlockSpec((BM, 384), lambda i: (i, 0)),
            pl.BlockSpec((384, 256), lambda i: (0, 0)),
            pl.BlockSpec((1, 256), lambda i: (0, 0)),
        ],
        out_specs=[
            pl.BlockSpec((BM, 128), lambda i: (i, 0)),
            pl.BlockSpec((BM, 128), lambda i: (i, 0)),
        ],
        out_shape=[
            jax.ShapeDtypeStruct((MP, 128), F32),
            jax.ShapeDtypeStruct((MP, 128), F32),
        ],
    )(h, w, b.reshape(1, 256))


def _z_body(mu_ref, lv_ref, e_ref, z_ref):
    lv = 0.5 * lv_ref[...]
    std = jnp.where(lv > 0, lv, jnp.exp(lv) - 1.0) + (1.0 + 1e-5)
    z_ref[...] = mu_ref[...] + std * e_ref[...]


def _z_kernel(mu, lv, eps):
    return pl.pallas_call(
        _z_body,
        grid=(MP // BM,),
        in_specs=[pl.BlockSpec((BM, 128), lambda i: (i, 0))] * 3,
        out_specs=pl.BlockSpec((BM, 128), lambda i: (i, 0)),
        out_shape=jax.ShapeDtypeStruct((MP, 128), F32),
    )(mu, lv, eps)


def _make_sc(bucketize):
    mesh = plsc.VectorSubcoreMesh(core_axis_name="c", subcore_axis_name="s")
    out_type = [jax.ShapeDtypeStruct((NPAD, FD), F32)]
    if bucketize:
        out_type += [
            jax.ShapeDtypeStruct((NWK * NRJ * CAP,), I32),
            jax.ShapeDtypeStruct((NWK * NRJ * CAP,), I32),
            jax.ShapeDtypeStruct((NWK * NRJ * CAP,), F32),
            jax.ShapeDtypeStruct((NWK * NRJ * 16,), I32),
        ]
    scratch = [
        pltpu.VMEM((NPAD,), F32),
        pltpu.VMEM((SR,), F32),
        pltpu.VMEM((SR,), F32),
        pltpu.VMEM((16, SR), F32),
        pltpu.VMEM((2 * CL,), I32),
        pltpu.VMEM((2 * CL,), I32),
        pltpu.VMEM((2 * CL,), F32),
        pltpu.VMEM((CL,), I32),
        pltpu.VMEM((CL,), F32),
        pltpu.VMEM((CL,), I32),
        pltpu.VMEM((RW, FD), F32),
        pltpu.VMEM((2, G, FD // 2), I32),
        pltpu.VMEM((RW,), F32),
        pltpu.VMEM((16,), F32),
        pltpu.VMEM((16,), I32),
        pltpu.SemaphoreType.DMA((2,)),
        pltpu.SemaphoreType.DMA,
    ]
    if bucketize:
        for _ in range(NRJ):
            scratch += [pltpu.VMEM((LB,), I32), pltpu.VMEM((LB,), I32),
                        pltpu.VMEM((LB,), F32)]

    def body(*refs):
        if bucketize:
            (src_h, dst_h, ew_h, ssrc_h, sdst_h, sn_h, ae_h, hw_h,
             out_h, bs_h, bd_h, be_h, cnt_h,
             ssrc_t, sdst_l, sloc_t, sums_t, c1, c2, c3, ssan, asan, dsan,
             outblk, stg, snloc, aev, cbuf, sem, csem, *lbufs) = refs
            lsrc = [lbufs[3 * j] for j in range(NRJ)]
            ldst = [lbufs[3 * j + 1] for j in range(NRJ)]
            lew = [lbufs[3 * j + 2] for j in range(NRJ)]
        else:
            (bs_h, bd_h, be_h, cnt_h, ssrc_h, sdst_h, sn_h, ae_h, hw_h,
             out_h,
             ssrc_t, sdst_l, sloc_t, sums_t, c1, c2, c3, ssan, asan, dsan,
             outblk, stg, snloc, aev, cbuf, sem, csem) = refs

        wid = lax.axis_index("c") * NS + lax.axis_index("s")
        base = wid * SR
        lane = lax.iota(I32, 16)
        zv = jnp.zeros((16,), F32)

        pltpu.sync_copy(ssrc_h, ssrc_t)
        pltpu.sync_copy(ae_h, aev)
        ae = aev[...][0]
        pltpu.sync_copy(sdst_h.at[pl.ds(pl.multiple_of(base, SR), SR)], sdst_l)

        def _zs(i, _):
            for c in range(SR // 16):
                sums_t[i, pl.ds(c * 16, 16)] = zv
            return 0
        lax.fori_loop(0, 16, _zs, 0)

        if bucketize:
            def chunk_body(ci, carry):
                pltpu.sync_copy(src_h.at[pl.ds(pl.multiple_of(ci * CE, 8), CE)],
                                c1.at[pl.ds(0, CE)])
                pltpu.sync_copy(dst_h.at[pl.ds(pl.multiple_of(ci * CE, 8), CE)],
                                c2.at[pl.ds(0, CE)])
                pltpu.sync_copy(ew_h.at[pl.ds(pl.multiple_of(ci * CE, 8), CE)],
                                c3.at[pl.ds(0, CE)])

                def g_body(g, cy):
                    s16 = c1[pl.ds(g * 16, 16)]
                    d16 = c2[pl.ds(g * 16, 16)]
                    e16 = c3[pl.ds(g * 16, 16)]
                    cidx = d16 - base
                    match = (cidx >= 0) & (cidx < SR)
                    cidx_s = jnp.where(match, cidx, 0)
                    which = jnp.right_shift(cidx_s, 6)
                    sv = plsc.load_gather(ssrc_t, [s16])
                    dv = plsc.load_gather(sdst_l, [cidx_s])
                    ee = sv + dv + e16 * ae
                    ee = jnp.where(ee > 0, ee, ee * 0.2)
                    ex = jnp.exp(ee)
                    plsc.addupdate_scatter(sums_t, [lane, cidx_s], ex,
                                           mask=match)
                    out = []
                    for j in range(NRJ):
                        fj, wj = cy[j], cy[NRJ + j]
                        mj = match & (which == j)
                        plsc.store_compressed(lsrc[j].at[pl.ds(fj, 16)], s16,
                                              mask=mj)
                        plsc.store_compressed(ldst[j].at[pl.ds(fj, 16)], d16,
                                              mask=mj)
                        plsc.store_compressed(lew[j].at[pl.ds(fj, 16)], e16,
                                              mask=mj)
                        fj = fj + plsc.all_reduce_population_count(mj)[0]
                        do = fj >= FB

                        @pl.when(do)
                        def _(j=j, wj=wj):
                            row = wid * NRJ + j
                            pltpu.sync_copy(
                                lsrc[j].at[pl.ds(0, FB)],
                                bs_h.at[pl.ds(pl.multiple_of(row * CAP + wj, FB), FB)])
                            pltpu.sync_copy(
                                ldst[j].at[pl.ds(0, FB)],
                                bd_h.at[pl.ds(pl.multiple_of(row * CAP + wj, FB), FB)])
                            pltpu.sync_copy(
                                lew[j].at[pl.ds(0, FB)],
                                be_h.at[pl.ds(pl.multiple_of(row * CAP + wj, FB), FB)])
                            t1 = lsrc[j][pl.ds(FB, 16)]
                            lsrc[j][pl.ds(0, 16)] = t1
                            t2 = ldst[j][pl.ds(FB, 16)]
                            ldst[j][pl.ds(0, 16)] = t2
                            t3 = lew[j][pl.ds(FB, 16)]
                            lew[j][pl.ds(0, 16)] = t3
                        out.append((jnp.where(do, fj - FB, fj),
                                    jnp.where(do, wj + FB, wj)))
                    return tuple([o[0] for o in out] + [o[1] for o in out])
                return lax.fori_loop(0, CE // 16, g_body, carry)

            z0 = jnp.zeros((), I32)
            fw = lax.fori_loop(0, E // CE, chunk_body, (z0,) * (2 * NRJ))
            for j in range(NRJ):
                fj, wj = fw[j], fw[NRJ + j]
                row = wid * NRJ + j
                pltpu.sync_copy(lsrc[j].at[pl.ds(0, FB)],
                                bs_h.at[pl.ds(pl.multiple_of(row * CAP + wj, FB), FB)])
                pltpu.sync_copy(ldst[j].at[pl.ds(0, FB)],
                                bd_h.at[pl.ds(pl.multiple_of(row * CAP + wj, FB), FB)])
                pltpu.sync_copy(lew[j].at[pl.ds(0, FB)],
                                be_h.at[pl.ds(pl.multiple_of(row * CAP + wj, FB), FB)])
                pltpu.sync_copy(lsrc[j].at[pl.ds(FB, 32)],
                                bs_h.at[pl.ds(pl.multiple_of(row * CAP + wj + FB, 8), 32)])
                pltpu.sync_copy(ldst[j].at[pl.ds(FB, 32)],
                                bd_h.at[pl.ds(pl.multiple_of(row * CAP + wj + FB, 8), 32)])
                pltpu.sync_copy(lew[j].at[pl.ds(FB, 32)],
                                be_h.at[pl.ds(pl.multiple_of(row * CAP + wj + FB, 8), 32)])
                cbuf[...] = jnp.zeros((16,), I32) + (wj + fj)
                pltpu.sync_copy(cbuf,
                                cnt_h.at[pl.ds(pl.multiple_of(row * 16, 16), 16)])
        else:
            def p1j(j, _):
                r = wid * NRJ + j
                _ = jax.named_scope

                @pl.when(r * RW < N)
                def _():
                    pltpu.sync_copy(cnt_h.at[pl.ds(pl.multiple_of(r * 16, 16), 16)],
                                    cbuf)
                    cnt = cbuf[...][0]
                    nch = jnp.right_shift(cnt + (CL - 1), 10)

                    def ch_body(k, _):
                        pltpu.sync_copy(bs_h.at[pl.ds(pl.multiple_of(r * CAP + k * CL, CL), CL)],
                                        c1.at[pl.ds(0, CL)])
                        pltpu.sync_copy(bd_h.at[pl.ds(pl.multiple_of(r * CAP + k * CL, CL), CL)],
                                        c2.at[pl.ds(0, CL)])
                        pltpu.sync_copy(be_h.at[pl.ds(pl.multiple_of(r * CAP + k * CL, CL), CL)],
                                        c3.at[pl.ds(0, CL)])

                        def g_body(g, _):
                            s16 = c1[pl.ds(g * 16, 16)]
                            d16 = c2[pl.ds(g * 16, 16)]
                            e16 = c3[pl.ds(g * 16, 16)]
                            gi = k * CL + g * 16 + lane
                            mm = gi < cnt
                            s16 = jnp.where(mm, s16, 0)
                            cidx = jnp.where(mm, d16 - base, 0)
                            sv = plsc.load_gather(ssrc_t, [s16])
                            dv = plsc.load_gather(sdst_l, [cidx])
                            ee = sv + dv + e16 * ae
                            ee = jnp.where(ee > 0, ee, ee * 0.2)
                            ex = jnp.exp(ee)
                            plsc.addupdate_scatter(sums_t, [lane, cidx], ex,
                                                   mask=mm)
                            return 0
                        lax.fori_loop(0, CL // 16, g_body, 0)
                        return 0
                    lax.fori_loop(0, nch, ch_body, 0)
                return 0
            lax.fori_loop(0, NRJ, p1j, 0)

        def fold_body(jj, _):
            acc = sums_t[0, pl.ds(jj * 16, 16)]
            for l in range(1, 16):
                acc = acc + sums_t[l, pl.ds(jj * 16, 16)]
            sloc_t[pl.ds(jj * 16, 16)] = acc
            return 0
        lax.fori_loop(0, SR // 16, fold_body, 0)

        def p2j(j, _):
            r = wid * NRJ + j

            @pl.when(r * RW < N)
            def _():
                pltpu.sync_copy(cnt_h.at[pl.ds(pl.multiple_of(r * 16, 16), 16)],
                                cbuf)
                cnt = cbuf[...][0]
                nch = jnp.right_shift(cnt + (CL - 1), 10)

                def zb(i, _):
                    for c in range(NV):
                        outblk[i, pl.ds(c * 16, 16)] = zv
                    return 0
                lax.fori_loop(0, RW, zb, 0)

                def cfetch(k, p):
                    pltpu.async_copy(bs_h.at[pl.ds(pl.multiple_of(r * CAP + k * CL, CL), CL)],
                                     c1.at[pl.ds(pl.multiple_of(p * CL, CL), CL)], csem)
                    pltpu.async_copy(bd_h.at[pl.ds(pl.multiple_of(r * CAP + k * CL, CL), CL)],
                                     c2.at[pl.ds(pl.multiple_of(p * CL, CL), CL)], csem)
                    pltpu.async_copy(be_h.at[pl.ds(pl.multiple_of(r * CAP + k * CL, CL), CL)],
                                     c3.at[pl.ds(pl.multiple_of(p * CL, CL), CL)], csem)

                def cdrain(p):
                    pltpu.make_async_copy(bs_h.at[pl.ds(0, CL)], c1.at[pl.ds(pl.multiple_of(p * CL, CL), CL)],
                                          csem).wait()
                    pltpu.make_async_copy(bd_h.at[pl.ds(0, CL)], c2.at[pl.ds(pl.multiple_of(p * CL, CL), CL)],
                                          csem).wait()
                    pltpu.make_async_copy(be_h.at[pl.ds(0, CL)], c3.at[pl.ds(pl.multiple_of(p * CL, CL), CL)],
                                          csem).wait()

                @pl.when(nch > 0)
                def _():
                    cfetch(0, 0)

                def ch_body(k, _):
                    p = jnp.bitwise_and(k, 1)
                    cdrain(p)

                    @pl.when(k + 1 < nch)
                    def _():
                        cfetch(k + 1, 1 - p)

                    @plsc.parallel_loop(0, CL // 16)
                    def _(g):
                        s16 = c1[pl.ds(p * CL + g * 16, 16)]
                        d16 = c2[pl.ds(p * CL + g * 16, 16)]
                        e16 = c3[pl.ds(p * CL + g * 16, 16)]
                        gi = k * CL + g * 16 + lane
                        mm = gi < cnt
                        s16 = jnp.where(mm, s16, 0)
                        dloc = jnp.where(mm, jnp.bitwise_and(d16, RW - 1), 0)
                        cidx = dloc + j * RW
                        sv = plsc.load_gather(ssrc_t, [s16])
                        dv = plsc.load_gather(sdst_l, [cidx])
                        ee = sv + dv + e16 * ae
                        ee = jnp.where(ee > 0, ee, ee * 0.2)
                        ex = jnp.exp(ee)
                        den = plsc.load_gather(sloc_t, [cidx]) + 1e-9
                        al = jnp.where(mm, ex / den, 0.0)
                        ssan[pl.ds(g * 16, 16)] = s16
                        asan[pl.ds(g * 16, 16)] = al
                        dsan[pl.ds(g * 16, 16)] = dloc

                    def fire(g, b):
                        pltpu.async_copy(
                            hw_h.at[ssan.at[pl.ds(g * G, G)]], stg.at[b],
                            sem.at[b])

                    def drain(b):
                        pltpu.make_async_copy(
                            hw_h.at[ssan.at[pl.ds(0, G)]], stg.at[b],
                            sem.at[b]).wait()

                    for b in range(2):
                        fire(b, b)

                    def pipe(g, _):
                        b = jnp.bitwise_and(g, 1)
                        drain(b)

                        @plsc.parallel_loop(0, G // 16)
                        def _(q):
                            av16 = asan[pl.ds(g * G + q * 16, 16)]
                            dl16 = dsan[pl.ds(g * G + q * 16, 16)]
                            for ii in range(16):
                                dl = dl16[ii]
                                av = av16[ii]
                                for half in range(2):
                                    vals = []
                                    for cc in range(12):
                                        v16 = stg[b, q * 16 + ii,
                                                  pl.ds(half * 192 + cc * 16, 16)]
                                        v32 = plsc.bitcast(v16, jnp.bfloat16)
                                        lo, hi = plsc.unpack(
                                            v32,
                                            format=plsc.PackFormat.INTERLEAVED)
                                        vals.append((half * 384 + cc * 32,
                                                     av * lo))
                                        vals.append((half * 384 + cc * 32 + 16,
                                                     av * hi))
                                    for off, v in vals:
                                        plsc.addupdate(
                                            outblk.at[dl, pl.ds(off, 16)], v)

                        @pl.when(g + 2 < NGG)
                        def _():
                            fire(g + 2, b)
                        return 0
                    lax.fori_loop(0, NGG, pipe, 0)
                    return 0
                lax.fori_loop(0, nch, ch_body, 0)

                pltpu.sync_copy(sn_h.at[pl.ds(pl.multiple_of(r * RW, RW), RW)],
                                snloc)

                @plsc.parallel_loop(0, RW // 16)
                def _(q):
                    sn16 = snloc[pl.ds(q * 16, 16)]
                    for ii in range(16):
                        sn = sn16[ii]
                        for half in range(2):
                            vals = [outblk[q * 16 + ii,
                                           pl.ds((half * 24 + c) * 16, 16)] * sn
                                    for c in range(24)]
                            for c in range(24):
                                v = vals[c]
                                outblk[q * 16 + ii, pl.ds((half * 24 + c) * 16, 16)] = (
                                    jnp.where(v > 0, v, v * 0.2))
                pltpu.sync_copy(outblk,
                                out_h.at[pl.ds(pl.multiple_of(r * RW, RW), RW)])
            return 0
        lax.fori_loop(0, NRJ, p2j, 0)

    return pl.kernel(body, out_type=out_type, mesh=mesh, scratch_types=scratch,
                     compiler_params=pltpu.CompilerParams(
                         needs_layout_passes=False))


_sc_bucket = _make_sc(True)
_sc_reuse = _make_sc(False)


_PERM = [(p // 32) * 32 + ((p % 32) // 2) + (0 if p % 2 == 0 else 16)
         for p in range(FD)]


def _gat_mm(xp, W, a_s, a_d):
    D = W.shape[0]
    wp = jnp.zeros((FD, FD), F32).at[:D, :D].set(W.T)
    wa = jnp.zeros((FD, 128), F32).at[:D, 0].set(a_s).at[:D, 1].set(a_d)
    perm = jnp.array(_PERM, dtype=jnp.int32)
    hw, os_ = _mm_gat(xp, wp[:, perm], wa[perm, :])
    hw32 = jax.lax.bitcast_convert_type(hw.reshape(MP, FD // 2, 2), I32)
    return hw32, os_


def _svecs(os_):
    ssrc = jnp.pad(os_[:N, 0], (0, NPAD - N))
    sdst = jnp.pad(os_[:N, 1], (0, NPAD - N))
    return ssrc, sdst


def kernel(features, edge_index, e_w, snorm_n, snorm_e, labels, maps_emb, eps,
           emb_W, emb_b,
           enc0_W, enc0_as, enc0_ad, enc0_ae,
           pri0_W, pri0_as, pri0_ad, pri0_ae,
           enc1_W, enc1_as, enc1_ad, enc1_ae,
           pri1_W, pri1_as, pri1_ad, pri1_ae,
           dec_W, dec_as, dec_ad,
           menc_W1, menc_b1, menc_Wmu, menc_bmu, menc_Wlv, menc_blv,
           mpri_W1, mpri_b1, mpri_Wmu, mpri_bmu, mpri_Wlv, mpri_blv,
           mdec_W0, mdec_b0, mdec_W1, mdec_b1):
    src = edge_index[0]
    dst = edge_index[1]
    ew = e_w[:, 0]
    sn = jnp.pad(snorm_n[:, 0], (0, NPAD - N))
    gt = labels

    h_emb = _mm(_pad2(features, MP, 128), _pad2(emb_W.T, 128, 128), emb_b)[:N, :128]

    x = _pad2(jnp.concatenate([maps_emb, h_emb, gt], axis=-1), MP, FD)
    hw, os_ = _gat_mm(x, enc0_W, enc0_as, enc0_ad)
    ssrc, sdst = _svecs(os_)
    ae = jnp.full((16,), enc0_ae[0], F32)
    out, bs, bd, be, cnts = _sc_bucket(src, dst, ew, ssrc, sdst, sn, ae, hw)

    x = _pad2(out[:N], MP, FD)
    hw, os_ = _gat_mm(x, enc1_W, enc1_as, enc1_ad)
    ssrc, sdst = _svecs(os_)
    ae = jnp.full((16,), enc1_ae[0], F32)
    out = _sc_reuse(bs, bd, be, cnts, ssrc, sdst, sn, ae, hw)[0]

    h = _pad2(jnp.concatenate([out[:N, :651], gt], axis=-1), MP, FD)
    hid = _mm(h, _pad2(menc_W1.T, FD, 384), menc_b1, act="lrelu")
    wmulv = (jnp.zeros((384, 256), F32)
             .at[:menc_Wmu.shape[1], 0:25].set(menc_Wmu.T)
             .at[:menc_Wlv.shape[1], 128:153].set(menc_Wlv.T))
    bmulv = (jnp.zeros((256,), F32).at[0:25].set(menc_bmu)
             .at[128:153].set(menc_blv))
    mu, lv = _mm_mulv(hid, wmulv, bmulv)
    z = _z_kernel(mu, lv, _pad2(eps, MP, 128))[:N, :25]

    x = _pad2(jnp.concatenate([maps_emb, h_emb, z], axis=-1), MP, FD)
    hw, os_ = _gat_mm(x, dec_W, dec_as, dec_ad)
    ssrc, sdst = _svecs(os_)
    out = _sc_reuse(bs, bd, be, cnts, ssrc, sdst, sn,
                    jnp.zeros((16,), F32), hw)[0]

    hd = jnp.concatenate([out[:N, :665], z], axis=-1)
    h0 = _mm(_pad2(hd, MP, FD), _pad2(mdec_W0.T, FD, FD), mdec_b0,
             act="lrelu")
    pred = _mm(h0, _pad2(mdec_W1.T, FD, 128), mdec_b1)[:N, :12]
    return pred


# speedup vs baseline: 1.0859x; 1.0006x over previous
import functools
import jax
import jax.numpy as jnp
from jax import lax
from jax.experimental import pallas as pl
from jax.experimental.pallas import tpu as pltpu
from jax.experimental.pallas import tpu_sc as plsc

N = 10000
E = 320000
MP = 10240
RW = 64
NR = 157
NRJ = 5
NWK = 32
NS = 16
SR = NRJ * RW
NPAD = MP
FD = 768
NV = FD // 16
CAP = E + 2048
CE = 800
CL = 1024
G = 16
NGG = CL // G
FB = 256
LB = FB + 32
BM = 512
F32 = jnp.float32
I32 = jnp.int32


def _pad2(x, m, n):
    M, Nc = x.shape
    return jnp.pad(x, ((0, m - M), (0, n - Nc)))


def _mm_body(x_ref, w_ref, b_ref, o_ref, *, act):
    acc = jnp.dot(x_ref[...], w_ref[...], preferred_element_type=F32)
    acc = acc + b_ref[...]
    if act == "lrelu":
        acc = jnp.where(acc > 0, acc, 0.2 * acc)
    o_ref[...] = acc


def _mm(x, w, b=None, act=None):
    M, K = x.shape
    K2, Nc = w.shape
    assert K == K2 and M % BM == 0, (x.shape, w.shape)
    if b is None:
        b = jnp.zeros((Nc,), F32)
    b2 = jnp.pad(b, (0, Nc - b.shape[0])).reshape(1, Nc)
    return pl.pallas_call(
        functools.partial(_mm_body, act=act),
        grid=(M // BM,),
        in_specs=[
            pl.BlockSpec((BM, K), lambda i: (i, 0)),
            pl.BlockSpec((K, Nc), lambda i: (0, 0)),
            pl.BlockSpec((1, Nc), lambda i: (0, 0)),
        ],
        out_specs=pl.BlockSpec((BM, Nc), lambda i: (i, 0)),
        out_shape=jax.ShapeDtypeStruct((M, Nc), F32),
    )(x, w, b2)


def _mm_gat_body(x_ref, w_ref, wa_ref, o_ref, os_ref):
    acc = jnp.dot(x_ref[...], w_ref[...], preferred_element_type=F32)
    o_ref[...] = acc.astype(jnp.bfloat16)
    os_ref[...] = jnp.dot(acc, wa_ref[...], preferred_element_type=F32)


def _mm_gat(x, w, wa):
    return pl.pallas_call(
        _mm_gat_body,
        grid=(MP // BM,),
        in_specs=[
            pl.BlockSpec((BM, FD), lambda i: (i, 0)),
            pl.BlockSpec((FD, FD), lambda i: (0, 0)),
            pl.BlockSpec((FD, 128), lambda i: (0, 0)),
        ],
        out_specs=[
            pl.BlockSpec((BM, FD), lambda i: (i, 0)),
            pl.BlockSpec((BM, 128), lambda i: (i, 0)),
        ],
        out_shape=[
            jax.ShapeDtypeStruct((MP, FD), jnp.bfloat16),
            jax.ShapeDtypeStruct((MP, 128), F32),
        ],
    )(x, w, wa)


def _mulv_body(h_ref, w_ref, b_ref, mu_ref, lv_ref):
    acc = jnp.dot(h_ref[...], w_ref[...], preferred_element_type=F32) + b_ref[...]
    mu_ref[...] = acc[:, 0:128]
    lv_ref[...] = acc[:, 128:256]


def _mm_mulv(h, w, b):
    return pl.pallas_call(
        _mulv_body,
        grid=(MP // BM,),
        in_specs=[
            pl.BlockSpec((BM, 384), lambda i: (i, 0)),
            pl.BlockSpec((384, 256), lambda i: (0, 0)),
            pl.BlockSpec((1, 256), lambda i: (0, 0)),
        ],
        out_specs=[
            pl.BlockSpec((BM, 128), lambda i: (i, 0)),
            pl.BlockSpec((BM, 128), lambda i: (i, 0)),
        ],
        out_shape=[
            jax.ShapeDtypeStruct((MP, 128), F32),
            jax.ShapeDtypeStruct((MP, 128), F32),
        ],
    )(h, w, b.reshape(1, 256))


def _z_body(mu_ref, lv_ref, e_ref, z_ref):
    lv = 0.5 * lv_ref[...]
    std = jnp.where(lv > 0, lv, jnp.exp(lv) - 1.0) + (1.0 + 1e-5)
    z_ref[...] = mu_ref[...] + std * e_ref[...]


def _z_kernel(mu, lv, eps):
    return pl.pallas_call(
        _z_body,
        grid=(MP // BM,),
        in_specs=[pl.BlockSpec((BM, 128), lambda i: (i, 0))] * 3,
        out_specs=pl.BlockSpec((BM, 128), lambda i: (i, 0)),
        out_shape=jax.ShapeDtypeStruct((MP, 128), F32),
    )(mu, lv, eps)


def _make_sc(bucketize):
    mesh = plsc.VectorSubcoreMesh(core_axis_name="c", subcore_axis_name="s")
    out_type = [jax.ShapeDtypeStruct((NPAD, FD), F32)]
    if bucketize:
        out_type += [
            jax.ShapeDtypeStruct((NWK * NRJ * CAP,), I32),
            jax.ShapeDtypeStruct((NWK * NRJ * CAP,), I32),
            jax.ShapeDtypeStruct((NWK * NRJ * CAP,), F32),
            jax.ShapeDtypeStruct((NWK * NRJ * 16,), I32),
        ]
    scratch = [
        pltpu.VMEM((NPAD,), F32),
        pltpu.VMEM((SR,), F32),
        pltpu.VMEM((SR,), F32),
        pltpu.VMEM((16, SR), F32),
        pltpu.VMEM((2 * CL,), I32),
        pltpu.VMEM((2 * CL,), I32),
        pltpu.VMEM((2 * CL,), F32),
        pltpu.VMEM((CL,), I32),
        pltpu.VMEM((CL,), F32),
        pltpu.VMEM((CL,), I32),
        pltpu.VMEM((RW, FD), F32),
        pltpu.VMEM((2, G, FD // 2), I32),
        pltpu.VMEM((RW,), F32),
        pltpu.VMEM((16,), F32),
        pltpu.VMEM((16,), I32),
        pltpu.SemaphoreType.DMA((2,)),
        pltpu.SemaphoreType.DMA,
    ]
    if bucketize:
        for _ in range(NRJ):
            scratch += [pltpu.VMEM((LB,), I32), pltpu.VMEM((LB,), I32),
                        pltpu.VMEM((LB,), F32)]

    def body(*refs):
        if bucketize:
            (src_h, dst_h, ew_h, ssrc_h, sdst_h, sn_h, ae_h, hw_h,
             out_h, bs_h, bd_h, be_h, cnt_h,
             ssrc_t, sdst_l, sloc_t, sums_t, c1, c2, c3, ssan, asan, dsan,
             outblk, stg, snloc, aev, cbuf, sem, csem, *lbufs) = refs
            lsrc = [lbufs[3 * j] for j in range(NRJ)]
            ldst = [lbufs[3 * j + 1] for j in range(NRJ)]
            lew = [lbufs[3 * j + 2] for j in range(NRJ)]
        else:
            (bs_h, bd_h, be_h, cnt_h, ssrc_h, sdst_h, sn_h, ae_h, hw_h,
             out_h,
             ssrc_t, sdst_l, sloc_t, sums_t, c1, c2, c3, ssan, asan, dsan,
             outblk, stg, snloc, aev, cbuf, sem, csem) = refs

        wid = lax.axis_index("c") * NS + lax.axis_index("s")
        base = wid * SR
        lane = lax.iota(I32, 16)
        zv = jnp.zeros((16,), F32)

        pltpu.sync_copy(ssrc_h, ssrc_t)
        pltpu.sync_copy(ae_h, aev)
        ae = aev[...][0]
        pltpu.sync_copy(sdst_h.at[pl.ds(pl.multiple_of(base, SR), SR)], sdst_l)

        def _zs(i, _):
            for c in range(SR // 16):
                sums_t[i, pl.ds(c * 16, 16)] = zv
            return 0
        lax.fori_loop(0, 16, _zs, 0)

        if bucketize:
            def chunk_body(ci, carry):
                pltpu.sync_copy(src_h.at[pl.ds(pl.multiple_of(ci * CE, 8), CE)],
                                c1.at[pl.ds(0, CE)])
                pltpu.sync_copy(dst_h.at[pl.ds(pl.multiple_of(ci * CE, 8), CE)],
                                c2.at[pl.ds(0, CE)])
                pltpu.sync_copy(ew_h.at[pl.ds(pl.multiple_of(ci * CE, 8), CE)],
                                c3.at[pl.ds(0, CE)])

                def g_body(g, cy):
                    s16 = c1[pl.ds(g * 16, 16)]
                    d16 = c2[pl.ds(g * 16, 16)]
                    e16 = c3[pl.ds(g * 16, 16)]
                    cidx = d16 - base
                    match = (cidx >= 0) & (cidx < SR)
                    cidx_s = jnp.where(match, cidx, 0)
                    which = jnp.right_shift(cidx_s, 6)
                    sv = plsc.load_gather(ssrc_t, [s16])
                    dv = plsc.load_gather(sdst_l, [cidx_s])
                    ee = sv + dv + e16 * ae
                    ee = jnp.where(ee > 0, ee, ee * 0.2)
                    ex = jnp.exp(ee)
                    plsc.addupdate_scatter(sums_t, [lane, cidx_s], ex,
                                           mask=match)
                    out = []
                    for j in range(NRJ):
                        fj, wj = cy[j], cy[NRJ + j]
                        mj = match & (which == j)
                        plsc.store_compressed(lsrc[j].at[pl.ds(fj, 16)], s16,
                                              mask=mj)
                        plsc.store_compressed(ldst[j].at[pl.ds(fj, 16)], d16,
                                              mask=mj)
                        plsc.store_compressed(lew[j].at[pl.ds(fj, 16)], e16,
                                              mask=mj)
                        fj = fj + plsc.all_reduce_population_count(mj)[0]
                        do = fj >= FB

                        @pl.when(do)
                        def _(j=j, wj=wj):
                            row = wid * NRJ + j
                            pltpu.sync_copy(
                                lsrc[j].at[pl.ds(0, FB)],
                                bs_h.at[pl.ds(pl.multiple_of(row * CAP + wj, FB), FB)])
                            pltpu.sync_copy(
                                ldst[j].at[pl.ds(0, FB)],
                                bd_h.at[pl.ds(pl.multiple_of(row * CAP + wj, FB), FB)])
                            pltpu.sync_copy(
                                lew[j].at[pl.ds(0, FB)],
                                be_h.at[pl.ds(pl.multiple_of(row * CAP + wj, FB), FB)])
                            t1 = lsrc[j][pl.ds(FB, 16)]
                            lsrc[j][pl.ds(0, 16)] = t1
                            t2 = ldst[j][pl.ds(FB, 16)]
                            ldst[j][pl.ds(0, 16)] = t2
                            t3 = lew[j][pl.ds(FB, 16)]
                            lew[j][pl.ds(0, 16)] = t3
                        out.append((jnp.where(do, fj - FB, fj),
                                    jnp.where(do, wj + FB, wj)))
                    return tuple([o[0] for o in out] + [o[1] for o in out])
                return lax.fori_loop(0, CE // 16, g_body, carry)

            z0 = jnp.zeros((), I32)
            fw = lax.fori_loop(0, E // CE, chunk_body, (z0,) * (2 * NRJ))
            for j in range(NRJ):
                fj, wj = fw[j], fw[NRJ + j]
                row = wid * NRJ + j
                pltpu.sync_copy(lsrc[j].at[pl.ds(0, FB)],
                                bs_h.at[pl.ds(pl.multiple_of(row * CAP + wj, FB), FB)])
                pltpu.sync_copy(ldst[j].at[pl.ds(0, FB)],
                                bd_h.at[pl.ds(pl.multiple_of(row * CAP + wj, FB), FB)])
                pltpu.sync_copy(lew[j].at[pl.ds(0, FB)],
                                be_h.at[pl.ds(pl.multiple_of(row * CAP + wj, FB), FB)])
                pltpu.sync_copy(lsrc[j].at[pl.ds(FB, 32)],
                                bs_h.at[pl.ds(pl.multiple_of(row * CAP + wj + FB, 8), 32)])
                pltpu.sync_copy(ldst[j].at[pl.ds(FB, 32)],
                                bd_h.at[pl.ds(pl.multiple_of(row * CAP + wj + FB, 8), 32)])
                pltpu.sync_copy(lew[j].at[pl.ds(FB, 32)],
                                be_h.at[pl.ds(pl.multiple_of(row * CAP + wj + FB, 8), 32)])
                cbuf[...] = jnp.zeros((16,), I32) + (wj + fj)
                pltpu.sync_copy(cbuf,
                                cnt_h.at[pl.ds(pl.multiple_of(row * 16, 16), 16)])
        else:
            def p1j(j, _):
                r = wid * NRJ + j

                @pl.when(r * RW < N)
                def _():
                    pltpu.sync_copy(cnt_h.at[pl.ds(pl.multiple_of(r * 16, 16), 16)],
                                    cbuf)
                    cnt = cbuf[...][0]
                    nch = jnp.right_shift(cnt + (CL - 1), 10)

                    def ch_body(k, _):
                        pltpu.sync_copy(bs_h.at[pl.ds(pl.multiple_of(r * CAP + k * CL, CL), CL)],
                                        c1.at[pl.ds(0, CL)])
                        pltpu.sync_copy(bd_h.at[pl.ds(pl.multiple_of(r * CAP + k * CL, CL), CL)],
                                        c2.at[pl.ds(0, CL)])
                        pltpu.sync_copy(be_h.at[pl.ds(pl.multiple_of(r * CAP + k * CL, CL), CL)],
                                        c3.at[pl.ds(0, CL)])

                        def g_body(g, _):
                            s16 = c1[pl.ds(g * 16, 16)]
                            d16 = c2[pl.ds(g * 16, 16)]
                            e16 = c3[pl.ds(g * 16, 16)]
                            gi = k * CL + g * 16 + lane
                            mm = gi < cnt
                            s16 = jnp.where(mm, s16, 0)
                            cidx = jnp.where(mm, d16 - base, 0)
                            sv = plsc.load_gather(ssrc_t, [s16])
                            dv = plsc.load_gather(sdst_l, [cidx])
                            ee = sv + dv + e16 * ae
                            ee = jnp.where(ee > 0, ee, ee * 0.2)
                            ex = jnp.exp(ee)
                            plsc.addupdate_scatter(sums_t, [lane, cidx], ex,
                                                   mask=mm)
                            return 0
                        lax.fori_loop(0, CL // 16, g_body, 0)
                        return 0
                    lax.fori_loop(0, nch, ch_body, 0)
                return 0
            lax.fori_loop(0, NRJ, p1j, 0)

        def fold_body(jj, _):
            acc = sums_t[0, pl.ds(jj * 16, 16)]
            for l in range(1, 16):
                acc = acc + sums_t[l, pl.ds(jj * 16, 16)]
            sloc_t[pl.ds(jj * 16, 16)] = acc
            return 0
        lax.fori_loop(0, SR // 16, fold_body, 0)

        def p2j(j, _):
            r = wid * NRJ + j

            @pl.when(r * RW < N)
            def _():
                pltpu.sync_copy(cnt_h.at[pl.ds(pl.multiple_of(r * 16, 16), 16)],
                                cbuf)
                cnt = cbuf[...][0]
                nch = jnp.right_shift(cnt + (CL - 1), 10)

                def zb(i, _):
                    for c in range(NV):
                        outblk[i, pl.ds(c * 16, 16)] = zv
                    return 0
                lax.fori_loop(0, RW, zb, 0)

                def cfetch(k, p):
                    pltpu.async_copy(bs_h.at[pl.ds(pl.multiple_of(r * CAP + k * CL, CL), CL)],
                                     c1.at[pl.ds(pl.multiple_of(p * CL, CL), CL)], csem)
                    pltpu.async_copy(bd_h.at[pl.ds(pl.multiple_of(r * CAP + k * CL, CL), CL)],
                                     c2.at[pl.ds(pl.multiple_of(p * CL, CL), CL)], csem)
                    pltpu.async_copy(be_h.at[pl.ds(pl.multiple_of(r * CAP + k * CL, CL), CL)],
                                     c3.at[pl.ds(pl.multiple_of(p * CL, CL), CL)], csem)

                def cdrain(p):
                    pltpu.make_async_copy(bs_h.at[pl.ds(0, CL)], c1.at[pl.ds(pl.multiple_of(p * CL, CL), CL)],
                                          csem).wait()
                    pltpu.make_async_copy(bd_h.at[pl.ds(0, CL)], c2.at[pl.ds(pl.multiple_of(p * CL, CL), CL)],
                                          csem).wait()
                    pltpu.make_async_copy(be_h.at[pl.ds(0, CL)], c3.at[pl.ds(pl.multiple_of(p * CL, CL), CL)],
                                          csem).wait()

                @pl.when(nch > 0)
                def _():
                    cfetch(0, 0)

                def ch_body(k, _):
                    p = jnp.bitwise_and(k, 1)
                    cdrain(p)

                    @pl.when(k + 1 < nch)
                    def _():
                        cfetch(k + 1, 1 - p)

                    @plsc.parallel_loop(0, CL // 16)
                    def _(g):
                        s16 = c1[pl.ds(p * CL + g * 16, 16)]
                        d16 = c2[pl.ds(p * CL + g * 16, 16)]
                        e16 = c3[pl.ds(p * CL + g * 16, 16)]
                        gi = k * CL + g * 16 + lane
                        mm = gi < cnt
                        s16 = jnp.where(mm, s16, 0)
                        dloc = jnp.where(mm, jnp.bitwise_and(d16, RW - 1), 0)
                        cidx = dloc + j * RW
                        sv = plsc.load_gather(ssrc_t, [s16])
                        dv = plsc.load_gather(sdst_l, [cidx])
                        ee = sv + dv + e16 * ae
                        ee = jnp.where(ee > 0, ee, ee * 0.2)
                        ex = jnp.exp(ee)
                        den = plsc.load_gather(sloc_t, [cidx]) + 1e-9
                        al = jnp.where(mm, ex / den, 0.0)
                        ssan[pl.ds(g * 16, 16)] = s16
                        asan[pl.ds(g * 16, 16)] = al
                        dsan[pl.ds(g * 16, 16)] = dloc

                    def fire(g, b):
                        pltpu.async_copy(
                            hw_h.at[ssan.at[pl.ds(g * G, G)]], stg.at[b],
                            sem.at[b])

                    def drain(b):
                        pltpu.make_async_copy(
                            hw_h.at[ssan.at[pl.ds(0, G)]], stg.at[b],
                            sem.at[b]).wait()

                    for b in range(2):
                        fire(b, b)

                    def pipe(g, _):
                        b = jnp.bitwise_and(g, 1)
                        drain(b)

                        @plsc.parallel_loop(0, G // 16)
                        def _(q):
                            av16 = asan[pl.ds(g * G + q * 16, 16)]
                            dl16 = dsan[pl.ds(g * G + q * 16, 16)]
                            for ii in range(16):
                                dl = dl16[ii]
                                av = av16[ii]
                                for half in range(2):
                                    vals = []
                                    for cc in range(12):
                                        v16 = stg[b, q * 16 + ii,
                                                  pl.ds(half * 192 + cc * 16, 16)]
                                        v32 = plsc.bitcast(v16, jnp.bfloat16)
                                        lo, hi = plsc.unpack(
                                            v32,
                                            format=plsc.PackFormat.INTERLEAVED)
                                        vals.append((half * 384 + cc * 32,
                                                     av * lo))
                                        vals.append((half * 384 + cc * 32 + 16,
                                                     av * hi))
                                    for off, v in vals:
                                        plsc.addupdate(
                                            outblk.at[dl, pl.ds(off, 16)], v)

                        @pl.when(g + 2 < NGG)
                        def _():
                            fire(g + 2, b)
                        return 0
                    lax.fori_loop(0, NGG, pipe, 0)
                    return 0
                lax.fori_loop(0, nch, ch_body, 0)

                pltpu.sync_copy(sn_h.at[pl.ds(pl.multiple_of(r * RW, RW), RW)],
                                snloc)

                @plsc.parallel_loop(0, RW // 16)
                def _(q):
                    sn16 = snloc[pl.ds(q * 16, 16)]
                    for ii in range(16):
                        sn = sn16[ii]
                        for half in range(2):
                            vals = [outblk[q * 16 + ii,
                                           pl.ds((half * 24 + c) * 16, 16)] * sn
                                    for c in range(24)]
                            for c in range(24):
                                v = vals[c]
                                outblk[q * 16 + ii, pl.ds((half * 24 + c) * 16, 16)] = (
                                    jnp.where(v > 0, v, v * 0.2))
                pltpu.sync_copy(outblk,
                                out_h.at[pl.ds(pl.multiple_of(r * RW, RW), RW)])
            return 0
        lax.fori_loop(0, NRJ, p2j, 0)

    return pl.kernel(body, out_type=out_type, mesh=mesh, scratch_types=scratch,
                     compiler_params=pltpu.CompilerParams(
                         needs_layout_passes=False))


_sc_bucket = _make_sc(True)
_sc_reuse = _make_sc(False)


_PERM = [(p // 32) * 32 + ((p % 32) // 2) + (0 if p % 2 == 0 else 16)
         for p in range(FD)]


def _gat_mm(xp, W, a_s, a_d):
    D = W.shape[0]
    wp = jnp.zeros((FD, FD), F32).at[:D, :D].set(W.T)
    wa = jnp.zeros((FD, 128), F32).at[:D, 0].set(a_s).at[:D, 1].set(a_d)
    perm = jnp.array(_PERM, dtype=jnp.int32)
    hw, os_ = _mm_gat(xp, wp[:, perm], wa[perm, :])
    hw32 = jax.lax.bitcast_convert_type(hw.reshape(MP, FD // 2, 2), I32)
    return hw32, os_


def _svecs(os_):
    ssrc = jnp.pad(os_[:N, 0], (0, NPAD - N))
    sdst = jnp.pad(os_[:N, 1], (0, NPAD - N))
    return ssrc, sdst


def kernel(features, edge_index, e_w, snorm_n, snorm_e, labels, maps_emb, eps,
           emb_W, emb_b,
           enc0_W, enc0_as, enc0_ad, enc0_ae,
           pri0_W, pri0_as, pri0_ad, pri0_ae,
           enc1_W, enc1_as, enc1_ad, enc1_ae,
           pri1_W, pri1_as, pri1_ad, pri1_ae,
           dec_W, dec_as, dec_ad,
           menc_W1, menc_b1, menc_Wmu, menc_bmu, menc_Wlv, menc_blv,
           mpri_W1, mpri_b1, mpri_Wmu, mpri_bmu, mpri_Wlv, mpri_blv,
           mdec_W0, mdec_b0, mdec_W1, mdec_b1):
    src = edge_index[0]
    dst = edge_index[1]
    ew = e_w[:, 0]
    sn = jnp.pad(snorm_n[:, 0], (0, NPAD - N))
    gt = labels

    h_emb = _mm(_pad2(features, MP, 128), _pad2(emb_W.T, 128, 128), emb_b)[:N, :128]

    x = _pad2(jnp.concatenate([maps_emb, h_emb, gt], axis=-1), MP, FD)
    hw, os_ = _gat_mm(x, enc0_W, enc0_as, enc0_ad)
    ssrc, sdst = _svecs(os_)
    ae = jnp.full((16,), enc0_ae[0], F32)
    out, bs, bd, be, cnts = _sc_bucket(src, dst, ew, ssrc, sdst, sn, ae, hw)

    x = _pad2(out[:N], MP, FD)
    hw, os_ = _gat_mm(x, enc1_W, enc1_as, enc1_ad)
    ssrc, sdst = _svecs(os_)
    ae = jnp.full((16,), enc1_ae[0], F32)
    out = _sc_reuse(bs, bd, be, cnts, ssrc, sdst, sn, ae, hw)[0]

    h = _pad2(jnp.concatenate([out[:N, :651], gt], axis=-1), MP, FD)
    hid = _mm(h, _pad2(menc_W1.T, FD, 384), menc_b1, act="lrelu")
    wmulv = (jnp.zeros((384, 256), F32)
             .at[:menc_Wmu.shape[1], 0:25].set(menc_Wmu.T)
             .at[:menc_Wlv.shape[1], 128:153].set(menc_Wlv.T))
    bmulv = (jnp.zeros((256,), F32).at[0:25].set(menc_bmu)
             .at[128:153].set(menc_blv))
    mu, lv = _mm_mulv(hid, wmulv, bmulv)
    z = _z_kernel(mu, lv, _pad2(eps, MP, 128))[:N, :25]

    x = _pad2(jnp.concatenate([maps_emb, h_emb, z], axis=-1), MP, FD)
    hw, os_ = _gat_mm(x, dec_W, dec_as, dec_ad)
    ssrc, sdst = _svecs(os_)
    out = _sc_reuse(bs, bd, be, cnts, ssrc, sdst, sn,
                    jnp.zeros((16,), F32), hw)[0]

    hd = jnp.concatenate([out[:N, :665], z], axis=-1)
    h0 = _mm(_pad2(hd, MP, FD), _pad2(mdec_W0.T, FD, FD), mdec_b0,
             act="lrelu")
    pred = _mm(h0, _pad2(mdec_W1.T, FD, 128), mdec_b1)[:N, :12]
    return pred


# speedup vs baseline: 1.0958x; 1.0091x over previous
import functools
import jax
import jax.numpy as jnp
from jax import lax
from jax.experimental import pallas as pl
from jax.experimental.pallas import tpu as pltpu
from jax.experimental.pallas import tpu_sc as plsc

N = 10000
E = 320000
MP = 10240
RW = 64
NR = 157
NRJ = 5
NWK = 32
NS = 16
SR = NRJ * RW
NPAD = MP
FD = 768
NV = FD // 16
CAP = E + 2048
CE = 800
CL = 1024
G = 16
NGG = CL // G
FB = 256
LB = FB + 32
BM = 512
F32 = jnp.float32
I32 = jnp.int32


def _pad2(x, m, n):
    M, Nc = x.shape
    return jnp.pad(x, ((0, m - M), (0, n - Nc)))


def _mm_body(x_ref, w_ref, b_ref, o_ref, *, act):
    acc = jnp.dot(x_ref[...], w_ref[...], preferred_element_type=F32)
    acc = acc + b_ref[...]
    if act == "lrelu":
        acc = jnp.where(acc > 0, acc, 0.2 * acc)
    o_ref[...] = acc


def _mm(x, w, b=None, act=None):
    M, K = x.shape
    K2, Nc = w.shape
    assert K == K2 and M % BM == 0, (x.shape, w.shape)
    if b is None:
        b = jnp.zeros((Nc,), F32)
    b2 = jnp.pad(b, (0, Nc - b.shape[0])).reshape(1, Nc)
    return pl.pallas_call(
        functools.partial(_mm_body, act=act),
        grid=(M // BM,),
        in_specs=[
            pl.BlockSpec((BM, K), lambda i: (i, 0)),
            pl.BlockSpec((K, Nc), lambda i: (0, 0)),
            pl.BlockSpec((1, Nc), lambda i: (0, 0)),
        ],
        out_specs=pl.BlockSpec((BM, Nc), lambda i: (i, 0)),
        out_shape=jax.ShapeDtypeStruct((M, Nc), F32),
    )(x, w, b2)


def _mm_gat_body(x_ref, w_ref, wa_ref, o_ref, os_ref):
    acc = jnp.dot(x_ref[...], w_ref[...], preferred_element_type=F32)
    o_ref[...] = acc.astype(jnp.bfloat16)
    os_ref[...] = jnp.dot(acc, wa_ref[...], preferred_element_type=F32)


def _mm_gat(x, w, wa):
    return pl.pallas_call(
        _mm_gat_body,
        grid=(MP // BM,),
        in_specs=[
            pl.BlockSpec((BM, FD), lambda i: (i, 0)),
            pl.BlockSpec((FD, FD), lambda i: (0, 0)),
            pl.BlockSpec((FD, 128), lambda i: (0, 0)),
        ],
        out_specs=[
            pl.BlockSpec((BM, FD), lambda i: (i, 0)),
            pl.BlockSpec((BM, 128), lambda i: (i, 0)),
        ],
        out_shape=[
            jax.ShapeDtypeStruct((MP, FD), jnp.bfloat16),
            jax.ShapeDtypeStruct((MP, 128), F32),
        ],
    )(x, w, wa)


def _mulv_body(h_ref, w_ref, b_ref, mu_ref, lv_ref):
    acc = jnp.dot(h_ref[...], w_ref[...], preferred_element_type=F32) + b_ref[...]
    mu_ref[...] = acc[:, 0:128]
    lv_ref[...] = acc[:, 128:256]


def _mm_mulv(h, w, b):
    return pl.pallas_call(
        _mulv_body,
        grid=(MP // BM,),
        in_specs=[
            pl.BlockSpec((BM, 384), lambda i: (i, 0)),
            pl.BlockSpec((384, 256), lambda i: (0, 0)),
            pl.BlockSpec((1, 256), lambda i: (0, 0)),
        ],
        out_specs=[
            pl.BlockSpec((BM, 128), lambda i: (i, 0)),
            pl.BlockSpec((BM, 128), lambda i: (i, 0)),
        ],
        out_shape=[
            jax.ShapeDtypeStruct((MP, 128), F32),
            jax.ShapeDtypeStruct((MP, 128), F32),
        ],
    )(h, w, b.reshape(1, 256))


def _z_body(mu_ref, lv_ref, e_ref, z_ref):
    lv = 0.5 * lv_ref[...]
    std = jnp.where(lv > 0, lv, jnp.exp(lv) - 1.0) + (1.0 + 1e-5)
    z_ref[...] = mu_ref[...] + std * e_ref[...]


def _z_kernel(mu, lv, eps):
    return pl.pallas_call(
        _z_body,
        grid=(MP // BM,),
        in_specs=[pl.BlockSpec((BM, 128), lambda i: (i, 0))] * 3,
        out_specs=pl.BlockSpec((BM, 128), lambda i: (i, 0)),
        out_shape=jax.ShapeDtypeStruct((MP, 128), F32),
    )(mu, lv, eps)


def _make_sc(bucketize):
    mesh = plsc.VectorSubcoreMesh(core_axis_name="c", subcore_axis_name="s")
    out_type = [jax.ShapeDtypeStruct((NPAD, FD), F32)]
    if bucketize:
        out_type += [
            jax.ShapeDtypeStruct((NWK * NRJ * CAP,), I32),
            jax.ShapeDtypeStruct((NWK * NRJ * CAP,), I32),
            jax.ShapeDtypeStruct((NWK * NRJ * CAP,), F32),
            jax.ShapeDtypeStruct((NWK * NRJ * 16,), I32),
        ]
    scratch = [
        pltpu.VMEM((NPAD,), F32),
        pltpu.VMEM((SR,), F32),
        pltpu.VMEM((SR,), F32),
        pltpu.VMEM((16, SR), F32),
        pltpu.VMEM((2 * CL,), I32),
        pltpu.VMEM((2 * CL,), I32),
        pltpu.VMEM((2 * CL,), F32),
        pltpu.VMEM((CL,), I32),
        pltpu.VMEM((CL,), F32),
        pltpu.VMEM((CL,), I32),
        pltpu.VMEM((RW, FD), F32),
        pltpu.VMEM((2, G, FD // 2), I32),
        pltpu.VMEM((RW,), F32),
        pltpu.VMEM((16,), F32),
        pltpu.VMEM((16,), I32),
        pltpu.SemaphoreType.DMA((2,)),
        pltpu.SemaphoreType.DMA,
    ]
    if bucketize:
        for _ in range(NRJ):
            scratch += [pltpu.VMEM((LB,), I32), pltpu.VMEM((LB,), I32),
                        pltpu.VMEM((LB,), F32)]

    def body(*refs):
        if bucketize:
            (src_h, dst_h, ew_h, ssrc_h, sdst_h, sn_h, ae_h, hw_h,
             out_h, bs_h, bd_h, be_h, cnt_h,
             ssrc_t, sdst_l, sloc_t, sums_t, c1, c2, c3, ssan, asan, dsan,
             outblk, stg, snloc, aev, cbuf, sem, csem, *lbufs) = refs
            lsrc = [lbufs[3 * j] for j in range(NRJ)]
            ldst = [lbufs[3 * j + 1] for j in range(NRJ)]
            lew = [lbufs[3 * j + 2] for j in range(NRJ)]
        else:
            (bs_h, bd_h, be_h, cnt_h, ssrc_h, sdst_h, sn_h, ae_h, hw_h,
             out_h,
             ssrc_t, sdst_l, sloc_t, sums_t, c1, c2, c3, ssan, asan, dsan,
             outblk, stg, snloc, aev, cbuf, sem, csem) = refs

        wid = lax.axis_index("c") * NS + lax.axis_index("s")
        base = wid * SR
        lane = lax.iota(I32, 16)
        zv = jnp.zeros((16,), F32)

        pltpu.sync_copy(ssrc_h, ssrc_t)
        pltpu.sync_copy(ae_h, aev)
        ae = aev[...][0]
        pltpu.sync_copy(sdst_h.at[pl.ds(pl.multiple_of(base, SR), SR)], sdst_l)

        def _zs(i, _):
            for c in range(SR // 16):
                sums_t[i, pl.ds(c * 16, 16)] = zv
            return 0
        lax.fori_loop(0, 16, _zs, 0)

        if bucketize:
            def chunk_body(ci, carry):
                pltpu.sync_copy(src_h.at[pl.ds(pl.multiple_of(ci * CE, 8), CE)],
                                c1.at[pl.ds(0, CE)])
                pltpu.sync_copy(dst_h.at[pl.ds(pl.multiple_of(ci * CE, 8), CE)],
                                c2.at[pl.ds(0, CE)])
                pltpu.sync_copy(ew_h.at[pl.ds(pl.multiple_of(ci * CE, 8), CE)],
                                c3.at[pl.ds(0, CE)])

                def g_body(g, cy):
                    s16 = c1[pl.ds(g * 16, 16)]
                    d16 = c2[pl.ds(g * 16, 16)]
                    e16 = c3[pl.ds(g * 16, 16)]
                    cidx = d16 - base
                    match = (cidx >= 0) & (cidx < SR)
                    cidx_s = jnp.where(match, cidx, 0)
                    which = jnp.right_shift(cidx_s, 6)
                    out = []
                    for j in range(NRJ):
                        fj, wj = cy[j], cy[NRJ + j]
                        mj = match & (which == j)
                        plsc.store_compressed(lsrc[j].at[pl.ds(fj, 16)], s16,
                                              mask=mj)
                        plsc.store_compressed(ldst[j].at[pl.ds(fj, 16)], d16,
                                              mask=mj)
                        plsc.store_compressed(lew[j].at[pl.ds(fj, 16)], e16,
                                              mask=mj)
                        fj = fj + plsc.all_reduce_population_count(mj)[0]
                        do = fj >= FB

                        @pl.when(do)
                        def _(j=j, wj=wj):
                            row = wid * NRJ + j
                            pltpu.sync_copy(
                                lsrc[j].at[pl.ds(0, FB)],
                                bs_h.at[pl.ds(pl.multiple_of(row * CAP + wj, FB), FB)])
                            pltpu.sync_copy(
                                ldst[j].at[pl.ds(0, FB)],
                                bd_h.at[pl.ds(pl.multiple_of(row * CAP + wj, FB), FB)])
                            pltpu.sync_copy(
                                lew[j].at[pl.ds(0, FB)],
                                be_h.at[pl.ds(pl.multiple_of(row * CAP + wj, FB), FB)])
                            t1 = lsrc[j][pl.ds(FB, 16)]
                            lsrc[j][pl.ds(0, 16)] = t1
                            t2 = ldst[j][pl.ds(FB, 16)]
                            ldst[j][pl.ds(0, 16)] = t2
                            t3 = lew[j][pl.ds(FB, 16)]
                            lew[j][pl.ds(0, 16)] = t3
                        out.append((jnp.where(do, fj - FB, fj),
                                    jnp.where(do, wj + FB, wj)))
                    return tuple([o[0] for o in out] + [o[1] for o in out])
                return lax.fori_loop(0, CE // 16, g_body, carry)

            z0 = jnp.zeros((), I32)
            fw = lax.fori_loop(0, E // CE, chunk_body, (z0,) * (2 * NRJ))
            for j in range(NRJ):
                fj, wj = fw[j], fw[NRJ + j]
                row = wid * NRJ + j
                pltpu.sync_copy(lsrc[j].at[pl.ds(0, FB)],
                                bs_h.at[pl.ds(pl.multiple_of(row * CAP + wj, FB), FB)])
                pltpu.sync_copy(ldst[j].at[pl.ds(0, FB)],
                                bd_h.at[pl.ds(pl.multiple_of(row * CAP + wj, FB), FB)])
                pltpu.sync_copy(lew[j].at[pl.ds(0, FB)],
                                be_h.at[pl.ds(pl.multiple_of(row * CAP + wj, FB), FB)])
                pltpu.sync_copy(lsrc[j].at[pl.ds(FB, 32)],
                                bs_h.at[pl.ds(pl.multiple_of(row * CAP + wj + FB, 8), 32)])
                pltpu.sync_copy(ldst[j].at[pl.ds(FB, 32)],
                                bd_h.at[pl.ds(pl.multiple_of(row * CAP + wj + FB, 8), 32)])
                pltpu.sync_copy(lew[j].at[pl.ds(FB, 32)],
                                be_h.at[pl.ds(pl.multiple_of(row * CAP + wj + FB, 8), 32)])
                cbuf[...] = jnp.zeros((16,), I32) + (wj + fj)
                pltpu.sync_copy(cbuf,
                                cnt_h.at[pl.ds(pl.multiple_of(row * 16, 16), 16)])
        if True:
            def p1j(j, _):
                r = wid * NRJ + j

                @pl.when(r * RW < N)
                def _():
                    pltpu.sync_copy(cnt_h.at[pl.ds(pl.multiple_of(r * 16, 16), 16)],
                                    cbuf)
                    cnt = cbuf[...][0]
                    nch = jnp.right_shift(cnt + (CL - 1), 10)

                    def ch_body(k, _):
                        pltpu.sync_copy(bs_h.at[pl.ds(pl.multiple_of(r * CAP + k * CL, CL), CL)],
                                        c1.at[pl.ds(0, CL)])
                        pltpu.sync_copy(bd_h.at[pl.ds(pl.multiple_of(r * CAP + k * CL, CL), CL)],
                                        c2.at[pl.ds(0, CL)])
                        pltpu.sync_copy(be_h.at[pl.ds(pl.multiple_of(r * CAP + k * CL, CL), CL)],
                                        c3.at[pl.ds(0, CL)])

                        def g_body(g, _):
                            s16 = c1[pl.ds(g * 16, 16)]
                            d16 = c2[pl.ds(g * 16, 16)]
                            e16 = c3[pl.ds(g * 16, 16)]
                            gi = k * CL + g * 16 + lane
                            mm = gi < cnt
                            s16 = jnp.where(mm, s16, 0)
                            cidx = jnp.where(mm, d16 - base, 0)
                            sv = plsc.load_gather(ssrc_t, [s16])
                            dv = plsc.load_gather(sdst_l, [cidx])
                            ee = sv + dv + e16 * ae
                            ee = jnp.where(ee > 0, ee, ee * 0.2)
                            ex = jnp.exp(ee)
                            plsc.addupdate_scatter(sums_t, [lane, cidx], ex,
                                                   mask=mm)
                            return 0
                        lax.fori_loop(0, CL // 16, g_body, 0)
                        return 0
                    lax.fori_loop(0, nch, ch_body, 0)
                return 0
            lax.fori_loop(0, NRJ, p1j, 0)

        def fold_body(jj, _):
            acc = sums_t[0, pl.ds(jj * 16, 16)]
            for l in range(1, 16):
                acc = acc + sums_t[l, pl.ds(jj * 16, 16)]
            sloc_t[pl.ds(jj * 16, 16)] = acc
            return 0
        lax.fori_loop(0, SR // 16, fold_body, 0)

        def p2j(j, _):
            r = wid * NRJ + j

            @pl.when(r * RW < N)
            def _():
                pltpu.sync_copy(cnt_h.at[pl.ds(pl.multiple_of(r * 16, 16), 16)],
                                cbuf)
                cnt = cbuf[...][0]
                nch = jnp.right_shift(cnt + (CL - 1), 10)

                def zb(i, _):
                    for c in range(NV):
                        outblk[i, pl.ds(c * 16, 16)] = zv
                    return 0
                lax.fori_loop(0, RW, zb, 0)

                def cfetch(k, p):
                    pltpu.async_copy(bs_h.at[pl.ds(pl.multiple_of(r * CAP + k * CL, CL), CL)],
                                     c1.at[pl.ds(pl.multiple_of(p * CL, CL), CL)], csem)
                    pltpu.async_copy(bd_h.at[pl.ds(pl.multiple_of(r * CAP + k * CL, CL), CL)],
                                     c2.at[pl.ds(pl.multiple_of(p * CL, CL), CL)], csem)
                    pltpu.async_copy(be_h.at[pl.ds(pl.multiple_of(r * CAP + k * CL, CL), CL)],
                                     c3.at[pl.ds(pl.multiple_of(p * CL, CL), CL)], csem)

                def cdrain(p):
                    pltpu.make_async_copy(bs_h.at[pl.ds(0, CL)], c1.at[pl.ds(pl.multiple_of(p * CL, CL), CL)],
                                          csem).wait()
                    pltpu.make_async_copy(bd_h.at[pl.ds(0, CL)], c2.at[pl.ds(pl.multiple_of(p * CL, CL), CL)],
                                          csem).wait()
                    pltpu.make_async_copy(be_h.at[pl.ds(0, CL)], c3.at[pl.ds(pl.multiple_of(p * CL, CL), CL)],
                                          csem).wait()

                @pl.when(nch > 0)
                def _():
                    cfetch(0, 0)

                def ch_body(k, _):
                    p = jnp.bitwise_and(k, 1)
                    cdrain(p)

                    @pl.when(k + 1 < nch)
                    def _():
                        cfetch(k + 1, 1 - p)

                    @plsc.parallel_loop(0, CL // 16)
                    def _(g):
                        s16 = c1[pl.ds(p * CL + g * 16, 16)]
                        d16 = c2[pl.ds(p * CL + g * 16, 16)]
                        e16 = c3[pl.ds(p * CL + g * 16, 16)]
                        gi = k * CL + g * 16 + lane
                        mm = gi < cnt
                        s16 = jnp.where(mm, s16, 0)
                        dloc = jnp.where(mm, jnp.bitwise_and(d16, RW - 1), 0)
                        cidx = dloc + j * RW
                        sv = plsc.load_gather(ssrc_t, [s16])
                        dv = plsc.load_gather(sdst_l, [cidx])
                        ee = sv + dv + e16 * ae
                        ee = jnp.where(ee > 0, ee, ee * 0.2)
                        ex = jnp.exp(ee)
                        den = plsc.load_gather(sloc_t, [cidx]) + 1e-9
                        al = jnp.where(mm, ex / den, 0.0)
                        ssan[pl.ds(g * 16, 16)] = s16
                        asan[pl.ds(g * 16, 16)] = al
                        dsan[pl.ds(g * 16, 16)] = dloc

                    def fire(g, b):
                        pltpu.async_copy(
                            hw_h.at[ssan.at[pl.ds(g * G, G)]], stg.at[b],
                            sem.at[b])

                    def drain(b):
                        pltpu.make_async_copy(
                            hw_h.at[ssan.at[pl.ds(0, G)]], stg.at[b],
                            sem.at[b]).wait()

                    for b in range(2):
                        fire(b, b)

                    def pipe(g, _):
                        b = jnp.bitwise_and(g, 1)
                        drain(b)

                        @plsc.parallel_loop(0, G // 16)
                        def _(q):
                            av16 = asan[pl.ds(g * G + q * 16, 16)]
                            dl16 = dsan[pl.ds(g * G + q * 16, 16)]
                            for ii in range(16):
                                dl = dl16[ii]
                                av = av16[ii]
                                for half in range(2):
                                    vals = []
                                    for cc in range(12):
                                        v16 = stg[b, q * 16 + ii,
                                                  pl.ds(half * 192 + cc * 16, 16)]
                                        v32 = plsc.bitcast(v16, jnp.bfloat16)
                                        lo, hi = plsc.unpack(
                                            v32,
                                            format=plsc.PackFormat.INTERLEAVED)
                                        vals.append((half * 384 + cc * 32,
                                                     av * lo))
                                        vals.append((half * 384 + cc * 32 + 16,
                                                     av * hi))
                                    for off, v in vals:
                                        plsc.addupdate(
                                            outblk.at[dl, pl.ds(off, 16)], v)

                        @pl.when(g + 2 < NGG)
                        def _():
                            fire(g + 2, b)
                        return 0
                    lax.fori_loop(0, NGG, pipe, 0)
                    return 0
                lax.fori_loop(0, nch, ch_body, 0)

                pltpu.sync_copy(sn_h.at[pl.ds(pl.multiple_of(r * RW, RW), RW)],
                                snloc)

                @plsc.parallel_loop(0, RW // 16)
                def _(q):
                    sn16 = snloc[pl.ds(q * 16, 16)]
                    for ii in range(16):
                        sn = sn16[ii]
                        for half in range(2):
                            vals = [outblk[q * 16 + ii,
                                           pl.ds((half * 24 + c) * 16, 16)] * sn
                                    for c in range(24)]
                            for c in range(24):
                                v = vals[c]
                                outblk[q * 16 + ii, pl.ds((half * 24 + c) * 16, 16)] = (
                                    jnp.where(v > 0, v, v * 0.2))
                pltpu.sync_copy(outblk,
                                out_h.at[pl.ds(pl.multiple_of(r * RW, RW), RW)])
            return 0
        lax.fori_loop(0, NRJ, p2j, 0)

    return pl.kernel(body, out_type=out_type, mesh=mesh, scratch_types=scratch,
                     compiler_params=pltpu.CompilerParams(
                         needs_layout_passes=False))


_sc_bucket = _make_sc(True)
_sc_reuse = _make_sc(False)


_PERM = [(p // 32) * 32 + ((p % 32) // 2) + (0 if p % 2 == 0 else 16)
         for p in range(FD)]


def _gat_mm(xp, W, a_s, a_d):
    D = W.shape[0]
    wp = jnp.zeros((FD, FD), F32).at[:D, :D].set(W.T)
    wa = jnp.zeros((FD, 128), F32).at[:D, 0].set(a_s).at[:D, 1].set(a_d)
    perm = jnp.array(_PERM, dtype=jnp.int32)
    hw, os_ = _mm_gat(xp, wp[:, perm], wa[perm, :])
    hw32 = jax.lax.bitcast_convert_type(hw.reshape(MP, FD // 2, 2), I32)
    return hw32, os_


def _svecs(os_):
    ssrc = jnp.pad(os_[:N, 0], (0, NPAD - N))
    sdst = jnp.pad(os_[:N, 1], (0, NPAD - N))
    return ssrc, sdst


def kernel(features, edge_index, e_w, snorm_n, snorm_e, labels, maps_emb, eps,
           emb_W, emb_b,
           enc0_W, enc0_as, enc0_ad, enc0_ae,
           pri0_W, pri0_as, pri0_ad, pri0_ae,
           enc1_W, enc1_as, enc1_ad, enc1_ae,
           pri1_W, pri1_as, pri1_ad, pri1_ae,
           dec_W, dec_as, dec_ad,
           menc_W1, menc_b1, menc_Wmu, menc_bmu, menc_Wlv, menc_blv,
           mpri_W1, mpri_b1, mpri_Wmu, mpri_bmu, mpri_Wlv, mpri_blv,
           mdec_W0, mdec_b0, mdec_W1, mdec_b1):
    src = edge_index[0]
    dst = edge_index[1]
    ew = e_w[:, 0]
    sn = jnp.pad(snorm_n[:, 0], (0, NPAD - N))
    gt = labels

    h_emb = _mm(_pad2(features, MP, 128), _pad2(emb_W.T, 128, 128), emb_b)[:N, :128]

    x = _pad2(jnp.concatenate([maps_emb, h_emb, gt], axis=-1), MP, FD)
    hw, os_ = _gat_mm(x, enc0_W, enc0_as, enc0_ad)
    ssrc, sdst = _svecs(os_)
    ae = jnp.full((16,), enc0_ae[0], F32)
    out, bs, bd, be, cnts = _sc_bucket(src, dst, ew, ssrc, sdst, sn, ae, hw)

    x = _pad2(out[:N], MP, FD)
    hw, os_ = _gat_mm(x, enc1_W, enc1_as, enc1_ad)
    ssrc, sdst = _svecs(os_)
    ae = jnp.full((16,), enc1_ae[0], F32)
    out = _sc_reuse(bs, bd, be, cnts, ssrc, sdst, sn, ae, hw)[0]

    h = _pad2(jnp.concatenate([out[:N, :651], gt], axis=-1), MP, FD)
    hid = _mm(h, _pad2(menc_W1.T, FD, 384), menc_b1, act="lrelu")
    wmulv = (jnp.zeros((384, 256), F32)
             .at[:menc_Wmu.shape[1], 0:25].set(menc_Wmu.T)
             .at[:menc_Wlv.shape[1], 128:153].set(menc_Wlv.T))
    bmulv = (jnp.zeros((256,), F32).at[0:25].set(menc_bmu)
             .at[128:153].set(menc_blv))
    mu, lv = _mm_mulv(hid, wmulv, bmulv)
    z = _z_kernel(mu, lv, _pad2(eps, MP, 128))[:N, :25]

    x = _pad2(jnp.concatenate([maps_emb, h_emb, z], axis=-1), MP, FD)
    hw, os_ = _gat_mm(x, dec_W, dec_as, dec_ad)
    ssrc, sdst = _svecs(os_)
    out = _sc_reuse(bs, bd, be, cnts, ssrc, sdst, sn,
                    jnp.zeros((16,), F32), hw)[0]

    hd = jnp.concatenate([out[:N, :665], z], axis=-1)
    h0 = _mm(_pad2(hd, MP, FD), _pad2(mdec_W0.T, FD, FD), mdec_b0,
             act="lrelu")
    pred = _mm(h0, _pad2(mdec_W1.T, FD, 128), mdec_b1)[:N, :12]
    return pred
